# baseline (speedup 1.0000x reference)
import functools

import jax
import jax.numpy as jnp
from jax import lax
from jax.experimental import pallas as pl
from jax.experimental.pallas import tpu as pltpu
from jax.experimental.pallas import tpu_sc as plsc

N = 50000
E = 800000
HID = 64
WIN = 8
IN_CH = 8

EPAD = 819200
EROWS = EPAD // 128
NPAD = 50176
HALF = 25000
ACC_ROWS = 25600
DUMP_BASE = 25088
PAD_DST = NPAD - 64

BLK = 1000


def _sc_degree(dst2d, zeros_deg, ones_blk):
    mesh = plsc.VectorSubcoreMesh(core_axis_name="c", subcore_axis_name="s")
    RPC = EROWS // 2
    RPT = RPC // 16
    ZRT = NPAD // 16
    W = 8

    @functools.partial(
        pl.kernel,
        out_type=jax.ShapeDtypeStruct((2, NPAD, 16), jnp.float32),
        mesh=mesh,
        scratch_types=[
            pltpu.VMEM((RPT, 128), jnp.int32),
            pltpu.VMEM((128, 16), jnp.float32),
            pltpu.VMEM_SHARED((NPAD, 16), jnp.float32),
            pltpu.SemaphoreType.DMA,
        ],
        compiler_params=pltpu.CompilerParams(use_tc_tiling_on_sc=False),
    )
    def k(dst_hbm, z_hbm, ones_hbm, out_hbm, idx_v, ones_v, acc_sh, sem):
        cid = lax.axis_index("c")
        sid = lax.axis_index("s")
        tid = cid * 16 + sid
        pltpu.sync_copy(dst_hbm.at[pl.ds(tid * RPT, RPT)], idx_v)
        pltpu.sync_copy(ones_hbm, ones_v)
        pltpu.sync_copy(z_hbm.at[pl.ds(sid * ZRT, ZRT)],
                        acc_sh.at[pl.ds(sid * ZRT, ZRT)])
        plsc.subcore_barrier()

        @pl.loop(0, RPT)
        def _(j):
            pltpu.async_copy(ones_v, acc_sh.at[idx_v.at[j]], sem, add=True)

            @pl.when(j >= W)
            def _():
                pltpu.make_async_copy(ones_v, acc_sh.at[idx_v.at[0]], sem).wait()

        for _ in range(W):
            pltpu.make_async_copy(ones_v, acc_sh.at[idx_v.at[0]], sem).wait()
        plsc.subcore_barrier()
        pltpu.sync_copy(acc_sh.at[pl.ds(sid * ZRT, ZRT)],
                        out_hbm.at[cid, pl.ds(sid * ZRT, ZRT)])

    return k(dst2d, zeros_deg, ones_blk)


def _sc_gather_scatter(yt, src2d, dst2d, zeros_acc):
    mesh = plsc.VectorSubcoreMesh(core_axis_name="c", subcore_axis_name="s")
    RPT = EROWS // 16
    CH = 10
    NCH = RPT // CH
    GW = 64
    NBUF = 5
    D = 3
    SPC = 2 * CH
    STEPS = RPT * 2
    ZRT = ACC_ROWS // 16

    @functools.partial(
        pl.kernel,
        out_type=jax.ShapeDtypeStruct((2, ACC_ROWS, HID), jnp.float32),
        mesh=mesh,
        scratch_types=[
            [pltpu.VMEM((CH, 128), jnp.int32) for _ in range(2)],
            [pltpu.VMEM((CH, 128), jnp.int32) for _ in range(2)],
            [pltpu.VMEM((1, GW), jnp.int32) for _ in range(NBUF)],
            [pltpu.VMEM((1, GW), jnp.int32) for _ in range(NBUF)],
            [pltpu.VMEM((GW, HID), jnp.float32) for _ in range(NBUF)],
            pltpu.VMEM_SHARED((ACC_ROWS, HID), jnp.float32),
            [pltpu.SemaphoreType.DMA for _ in range(2)],
            [pltpu.SemaphoreType.DMA for _ in range(NBUF)],
            [pltpu.SemaphoreType.DMA for _ in range(NBUF)],
        ],
        compiler_params=pltpu.CompilerParams(use_tc_tiling_on_sc=False),
    )
    def k(yt_hbm, src_hbm, dst_hbm, z_hbm, out_hbm,
          src_c, dst_c, srow, drow, rows_v, acc_sh, isem, gsem, ssem):
        cid = lax.axis_index("c")
        sid = lax.axis_index("s")
        lo = cid * HALF
        base = sid * RPT
        pltpu.sync_copy(z_hbm.at[pl.ds(sid * ZRT, ZRT)],
                        acc_sh.at[pl.ds(sid * ZRT, ZRT)])
        plsc.subcore_barrier()

        lo_v = jnp.full((16,), lo, jnp.int32)
        hi_v = jnp.full((16,), lo + HALF, jnp.int32)
        lane = lax.iota(jnp.int32, 16)
        dump_vs = [DUMP_BASE + (sid & 7) * 64 + kk * 16 + lane
                   for kk in range(4)]

        def start_chunk_load(ck, p):
            pltpu.async_copy(src_hbm.at[pl.ds(base + ck * CH, CH)],
                             src_c[p], isem[p])
            pltpu.async_copy(dst_hbm.at[pl.ds(base + ck * CH, CH)],
                             dst_c[p], isem[p])

        def wait_chunk_load(ck, p):
            pltpu.make_async_copy(src_hbm.at[pl.ds(base + ck * CH, CH)],
                                  src_c[p], isem[p]).wait()
            pltpu.make_async_copy(dst_hbm.at[pl.ds(base + ck * CH, CH)],
                                  dst_c[p], isem[p]).wait()

        def wait_scatter(q):
            pltpu.make_async_copy(rows_v[q], acc_sh.at[drow[q].at[0]],
                                  ssem[q]).wait()

        def issue_scatter(q):
            pltpu.async_copy(rows_v[q], acc_sh.at[drow[q].at[0]], ssem[q],
                             add=True)

        def wait_gather(q):
            pltpu.make_async_copy(yt_hbm.at[srow[q].at[0]], rows_v[q],
                                  gsem[q]).wait()

        def half_step(p, jj, hh, s, q):
            pass

            for kk in range(4):
                so = pl.ds(kk * 16, 16)
                si = pl.ds(hh * GW + kk * 16, 16)
                srow[q].at[0][so] = src_c[p].at[jj][si]
                v = dst_c[p].at[jj][si]
                inr = (v >= lo_v) & (v < hi_v)
                drow[q].at[0][so] = jnp.where(inr, v - lo_v, dump_vs[kk])

            pltpu.async_copy(yt_hbm.at[srow[q].at[0]], rows_v[q], gsem[q])

            q2 = (q - D) % NBUF
            @pl.when(s >= D)
            def _():
                wait_gather(q2)

        start_chunk_load(0, 0)
        start_chunk_load(1, 1)

        @pl.loop(0, NCH // 2)
        def _(m):
            for c0 in range(2):
                ck = 2 * m + c0
                wait_chunk_load(ck, c0)

                @pl.loop(0, SPC // 10)
                def _(t):
                    for u in range(10):
                        jj = 5 * t + u // 2
                        half_step(c0, jj, u % 2, ck * SPC + 10 * t + u,
                                  u % NBUF)

                @pl.when(m < NCH // 2 - 1)
                def _():
                    start_chunk_load(ck + 2, c0)

        for i in range(D):
            q2 = (STEPS - D + i) % NBUF
            wait_gather(q2)
        plsc.subcore_barrier()
        pltpu.sync_copy(acc_sh.at[pl.ds(sid * ZRT, ZRT)],
                        out_hbm.at[cid, pl.ds(sid * ZRT, ZRT)])

    return k(yt, src2d, dst2d, zeros_acc)


def _tc_lstm(x2, wih_t, whh_t, bias, wg_t, deg2):
    def body(x_ref, wih_ref, whh_ref, b_ref, wg_ref, deg_ref, yt_ref):
        xb = x_ref[...]
        wih = wih_ref[...]
        whh = whh_ref[...]
        b = b_ref[...]
        h = jnp.zeros((BLK, HID), jnp.float32)
        c = jnp.zeros((BLK, HID), jnp.float32)
        for t in range(WIN):
            xt = xb[:, t * IN_CH:(t + 1) * IN_CH]
            g = jnp.dot(xt, wih, preferred_element_type=jnp.float32)
            g = g + jnp.dot(h, whh, preferred_element_type=jnp.float32) + b
            ig = jax.nn.sigmoid(g[:, 0:HID])
            fg = jax.nn.sigmoid(g[:, HID:2 * HID])
            gg = jnp.tanh(g[:, 2 * HID:3 * HID])
            og = jax.nn.sigmoid(g[:, 3 * HID:4 * HID])
            c = fg * c + ig * gg
            h = og * jnp.tanh(c)
        xt_out = jnp.dot(h, wg_ref[...], preferred_element_type=jnp.float32)
        yt_ref[...] = xt_out * lax.rsqrt(deg_ref[...])

    return pl.pallas_call(
        body,
        grid=(N // BLK,),
        in_specs=[
            pl.BlockSpec((BLK, WIN * IN_CH), lambda i: (i, 0)),
            pl.BlockSpec((IN_CH, 4 * HID), lambda i: (0, 0)),
            pl.BlockSpec((HID, 4 * HID), lambda i: (0, 0)),
            pl.BlockSpec((1, 4 * HID), lambda i: (0, 0)),
            pl.BlockSpec((HID, HID), lambda i: (0, 0)),
            pl.BlockSpec((BLK, 1), lambda i: (i, 0)),
        ],
        out_specs=pl.BlockSpec((BLK, HID), lambda i: (i, 0)),
        out_shape=jax.ShapeDtypeStruct((N, HID), jnp.float32),
        compiler_params=pltpu.CompilerParams(
            dimension_semantics=("arbitrary",)),
    )(x2, wih_t, whh_t, bias, wg_t, deg2)


def _tc_head(agg, yt, deg2, bgc, w1_t, b1r, w2r, b2r):
    def body(a_ref, y_ref, d_ref, bg_ref, w1_ref, b1_ref, w2_ref, b2_ref,
             o_ref):
        dinv = lax.rsqrt(d_ref[...])
        g = dinv * (a_ref[...] + y_ref[...]) + bg_ref[...]
        g = jnp.maximum(g, 0.0)
        o1 = jnp.dot(g, w1_ref[...], preferred_element_type=jnp.float32)
        o1 = jnp.maximum(o1 + b1_ref[...], 0.0)
        o_ref[...] = jnp.sum(o1 * w2_ref[...], axis=1, keepdims=True) + b2_ref[...]

    return pl.pallas_call(
        body,
        grid=(N // BLK,),
        in_specs=[
            pl.BlockSpec((BLK, HID), lambda i: (i, 0)),
            pl.BlockSpec((BLK, HID), lambda i: (i, 0)),
            pl.BlockSpec((BLK, 1), lambda i: (i, 0)),
            pl.BlockSpec((1, HID), lambda i: (0, 0)),
            pl.BlockSpec((HID, HID // 2), lambda i: (0, 0)),
            pl.BlockSpec((1, HID // 2), lambda i: (0, 0)),
            pl.BlockSpec((1, HID // 2), lambda i: (0, 0)),
            pl.BlockSpec((1, 1), lambda i: (0, 0)),
        ],
        out_specs=pl.BlockSpec((BLK, 1), lambda i: (i, 0)),
        out_shape=jax.ShapeDtypeStruct((N, 1), jnp.float32),
        compiler_params=pltpu.CompilerParams(
            dimension_semantics=("arbitrary",)),
    )(agg, yt, deg2, bgc, w1_t, b1r, w2r, b2r)


def kernel(x, edge_index, W_ih, W_hh, b_ih, b_hh, W_gcn, b_gcn, W1, b1, W2, b2):
    src = edge_index[0].astype(jnp.int32)
    dst = edge_index[1].astype(jnp.int32)
    padn = EPAD - E
    src2d = jnp.concatenate(
        [src, jnp.zeros((padn,), jnp.int32)]).reshape(EROWS, 128)
    dst2d = jnp.concatenate(
        [dst, jnp.full((padn,), PAD_DST, jnp.int32)]).reshape(EROWS, 128)

    zeros_deg = jnp.zeros((NPAD, 16), jnp.float32)
    ones_blk = jnp.ones((128, 16), jnp.float32)
    degp = _sc_degree(dst2d, zeros_deg, ones_blk)
    deg2 = (degp[0, :N, 0] + degp[1, :N, 0] + 1.0).reshape(N, 1)

    x2 = x.reshape(N, WIN * IN_CH)
    bias = (b_ih + b_hh).reshape(1, 4 * HID)
    yt = _tc_lstm(x2, W_ih.T, W_hh.T, bias, W_gcn.T, deg2)

    zeros_acc = jnp.zeros((ACC_ROWS, HID), jnp.float32)
    aggp = _sc_gather_scatter(yt, src2d, dst2d, zeros_acc)
    agg = jnp.concatenate([aggp[0, :HALF], aggp[1, :HALF]], axis=0)

    out = _tc_head(agg, yt, deg2, b_gcn.reshape(1, HID), W1.T,
                   b1.reshape(1, HID // 2), W2.reshape(1, HID // 2),
                   b2.reshape(1, 1))
    return out.reshape(1, N, 1)


# speedup vs baseline: 1.0004x; 1.0004x over previous
import functools

import jax
import jax.numpy as jnp
from jax import lax
from jax.experimental import pallas as pl
from jax.experimental.pallas import tpu as pltpu
from jax.experimental.pallas import tpu_sc as plsc

N = 50000
E = 800000
HID = 64
WIN = 8
IN_CH = 8

EPAD = 819200
EROWS = EPAD // 128
NPAD = 50176
HALF = 25000
ACC_ROWS = 25600
DUMP_BASE = 25088
PAD_DST = NPAD - 64

BLK = 1000


def _sc_degree(dst2d, zeros_deg, ones_blk):
    mesh = plsc.VectorSubcoreMesh(core_axis_name="c", subcore_axis_name="s")
    RPC = EROWS // 2
    RPT = RPC // 16
    ZRT = NPAD // 16
    W = 8

    @functools.partial(
        pl.kernel,
        out_type=jax.ShapeDtypeStruct((2, NPAD, 16), jnp.float32),
        mesh=mesh,
        scratch_types=[
            pltpu.VMEM((RPT, 128), jnp.int32),
            pltpu.VMEM((128, 16), jnp.float32),
            pltpu.VMEM_SHARED((NPAD, 16), jnp.float32),
            pltpu.SemaphoreType.DMA,
        ],
        compiler_params=pltpu.CompilerParams(use_tc_tiling_on_sc=False),
    )
    def k(dst_hbm, z_hbm, ones_hbm, out_hbm, idx_v, ones_v, acc_sh, sem):
        cid = lax.axis_index("c")
        sid = lax.axis_index("s")
        tid = cid * 16 + sid
        pltpu.sync_copy(dst_hbm.at[pl.ds(tid * RPT, RPT)], idx_v)
        pltpu.sync_copy(ones_hbm, ones_v)
        pltpu.sync_copy(z_hbm.at[pl.ds(sid * ZRT, ZRT)],
                        acc_sh.at[pl.ds(sid * ZRT, ZRT)])
        plsc.subcore_barrier()

        @pl.loop(0, RPT)
        def _(j):
            pltpu.async_copy(ones_v, acc_sh.at[idx_v.at[j]], sem, add=True)

            @pl.when(j >= W)
            def _():
                pltpu.make_async_copy(ones_v, acc_sh.at[idx_v.at[0]], sem).wait()

        for _ in range(W):
            pltpu.make_async_copy(ones_v, acc_sh.at[idx_v.at[0]], sem).wait()
        plsc.subcore_barrier()
        pltpu.sync_copy(acc_sh.at[pl.ds(sid * ZRT, ZRT)],
                        out_hbm.at[cid, pl.ds(sid * ZRT, ZRT)])

    return k(dst2d, zeros_deg, ones_blk)


def _sc_gather_scatter(yt, src2d, dst2d, zeros_acc):
    mesh = plsc.VectorSubcoreMesh(core_axis_name="c", subcore_axis_name="s")
    RPT = EROWS // 16
    CH = 10
    NCH = RPT // CH
    GW = 64
    NBUF = 5
    D = 3
    SPC = 2 * CH
    STEPS = RPT * 2
    ZRT = ACC_ROWS // 16

    @functools.partial(
        pl.kernel,
        out_type=jax.ShapeDtypeStruct((2, ACC_ROWS, HID), jnp.float32),
        mesh=mesh,
        scratch_types=[
            [pltpu.VMEM((CH, 128), jnp.int32) for _ in range(2)],
            [pltpu.VMEM((CH, 128), jnp.int32) for _ in range(2)],
            [pltpu.VMEM((1, GW), jnp.int32) for _ in range(NBUF)],
            [pltpu.VMEM((1, GW), jnp.int32) for _ in range(NBUF)],
            [pltpu.VMEM((GW, HID), jnp.float32) for _ in range(NBUF)],
            pltpu.VMEM_SHARED((ACC_ROWS, HID), jnp.float32),
            [pltpu.SemaphoreType.DMA for _ in range(2)],
            [pltpu.SemaphoreType.DMA for _ in range(NBUF)],
            [pltpu.SemaphoreType.DMA for _ in range(NBUF)],
        ],
        compiler_params=pltpu.CompilerParams(use_tc_tiling_on_sc=False),
    )
    def k(yt_hbm, src_hbm, dst_hbm, z_hbm, out_hbm,
          src_c, dst_c, srow, drow, rows_v, acc_sh, isem, gsem, ssem):
        cid = lax.axis_index("c")
        sid = lax.axis_index("s")
        lo = cid * HALF
        base = sid * RPT
        pltpu.sync_copy(z_hbm.at[pl.ds(sid * ZRT, ZRT)],
                        acc_sh.at[pl.ds(sid * ZRT, ZRT)])
        plsc.subcore_barrier()

        lo_v = jnp.full((16,), lo, jnp.int32)
        hi_v = jnp.full((16,), lo + HALF, jnp.int32)
        lane = lax.iota(jnp.int32, 16)
        dump_vs = [DUMP_BASE + (sid & 7) * 64 + kk * 16 + lane
                   for kk in range(4)]

        def start_chunk_load(ck, p):
            pltpu.async_copy(src_hbm.at[pl.ds(base + ck * CH, CH)],
                             src_c[p], isem[p])
            pltpu.async_copy(dst_hbm.at[pl.ds(base + ck * CH, CH)],
                             dst_c[p], isem[p])

        def wait_chunk_load(ck, p):
            pltpu.make_async_copy(src_hbm.at[pl.ds(base + ck * CH, CH)],
                                  src_c[p], isem[p]).wait()
            pltpu.make_async_copy(dst_hbm.at[pl.ds(base + ck * CH, CH)],
                                  dst_c[p], isem[p]).wait()

        def wait_scatter(q):
            pltpu.make_async_copy(rows_v[q], acc_sh.at[drow[q].at[0]],
                                  ssem[q]).wait()

        def issue_scatter(q):
            pltpu.async_copy(rows_v[q], acc_sh.at[drow[q].at[0]], ssem[q],
                             add=True)

        def wait_gather(q):
            pltpu.make_async_copy(yt_hbm.at[srow[q].at[0]], rows_v[q],
                                  gsem[q]).wait()

        def half_step(p, jj, hh, s, q):
            @pl.when(s >= NBUF)
            def _():
                wait_scatter(q)

            for kk in range(4):
                so = pl.ds(kk * 16, 16)
                si = pl.ds(hh * GW + kk * 16, 16)
                srow[q].at[0][so] = src_c[p].at[jj][si]
                v = dst_c[p].at[jj][si]
                inr = (v >= lo_v) & (v < hi_v)
                drow[q].at[0][so] = jnp.where(inr, v - lo_v, dump_vs[kk])

            pltpu.async_copy(yt_hbm.at[srow[q].at[0]], rows_v[q], gsem[q])

            q2 = (q - D) % NBUF
            @pl.when(s >= D)
            def _():
                wait_gather(q2)
                issue_scatter(q2)

        start_chunk_load(0, 0)
        start_chunk_load(1, 1)

        @pl.loop(0, NCH // 2)
        def _(m):
            for c0 in range(2):
                ck = 2 * m + c0
                wait_chunk_load(ck, c0)

                @pl.loop(0, SPC // 10)
                def _(t):
                    for u in range(10):
                        jj = 5 * t + u // 2
                        half_step(c0, jj, u % 2, ck * SPC + 10 * t + u,
                                  u % NBUF)

                @pl.when(m < NCH // 2 - 1)
                def _():
                    start_chunk_load(ck + 2, c0)

        for i in range(D):
            q2 = (STEPS - D + i) % NBUF
            wait_gather(q2)
            issue_scatter(q2)
        for q in range(NBUF):
            wait_scatter(q)
        plsc.subcore_barrier()
        pltpu.sync_copy(acc_sh.at[pl.ds(sid * ZRT, ZRT)],
                        out_hbm.at[cid, pl.ds(sid * ZRT, ZRT)])

    return k(yt, src2d, dst2d, zeros_acc)


def _tc_lstm(x2, wih_t, whh_t, bias, wg_t, deg2):
    def body(x_ref, wih_ref, whh_ref, b_ref, wg_ref, deg_ref, yt_ref):
        xb = x_ref[...]
        wih = wih_ref[...]
        whh = whh_ref[...]
        b = b_ref[...]
        h = jnp.zeros((BLK, HID), jnp.float32)
        c = jnp.zeros((BLK, HID), jnp.float32)
        for t in range(WIN):
            xt = xb[:, t * IN_CH:(t + 1) * IN_CH]
            g = jnp.dot(xt, wih, preferred_element_type=jnp.float32)
            g = g + jnp.dot(h, whh, preferred_element_type=jnp.float32) + b
            ig = jax.nn.sigmoid(g[:, 0:HID])
            fg = jax.nn.sigmoid(g[:, HID:2 * HID])
            gg = jnp.tanh(g[:, 2 * HID:3 * HID])
            og = jax.nn.sigmoid(g[:, 3 * HID:4 * HID])
            c = fg * c + ig * gg
            h = og * jnp.tanh(c)
        xt_out = jnp.dot(h, wg_ref[...], preferred_element_type=jnp.float32)
        yt_ref[...] = xt_out * lax.rsqrt(deg_ref[...])

    return pl.pallas_call(
        body,
        grid=(N // BLK,),
        in_specs=[
            pl.BlockSpec((BLK, WIN * IN_CH), lambda i: (i, 0)),
            pl.BlockSpec((IN_CH, 4 * HID), lambda i: (0, 0)),
            pl.BlockSpec((HID, 4 * HID), lambda i: (0, 0)),
            pl.BlockSpec((1, 4 * HID), lambda i: (0, 0)),
            pl.BlockSpec((HID, HID), lambda i: (0, 0)),
            pl.BlockSpec((BLK, 1), lambda i: (i, 0)),
        ],
        out_specs=pl.BlockSpec((BLK, HID), lambda i: (i, 0)),
        out_shape=jax.ShapeDtypeStruct((N, HID), jnp.float32),
        compiler_params=pltpu.CompilerParams(
            dimension_semantics=("arbitrary",)),
    )(x2, wih_t, whh_t, bias, wg_t, deg2)


def _tc_head(agg, yt, deg2, bgc, w1_t, b1r, w2r, b2r):
    def body(a_ref, y_ref, d_ref, bg_ref, w1_ref, b1_ref, w2_ref, b2_ref,
             o_ref):
        dinv = lax.rsqrt(d_ref[...])
        g = dinv * (a_ref[...] + y_ref[...]) + bg_ref[...]
        g = jnp.maximum(g, 0.0)
        o1 = jnp.dot(g, w1_ref[...], preferred_element_type=jnp.float32)
        o1 = jnp.maximum(o1 + b1_ref[...], 0.0)
        o_ref[...] = jnp.sum(o1 * w2_ref[...], axis=1, keepdims=True) + b2_ref[...]

    return pl.pallas_call(
        body,
        grid=(N // BLK,),
        in_specs=[
            pl.BlockSpec((BLK, HID), lambda i: (i, 0)),
            pl.BlockSpec((BLK, HID), lambda i: (i, 0)),
            pl.BlockSpec((BLK, 1), lambda i: (i, 0)),
            pl.BlockSpec((1, HID), lambda i: (0, 0)),
            pl.BlockSpec((HID, HID // 2), lambda i: (0, 0)),
            pl.BlockSpec((1, HID // 2), lambda i: (0, 0)),
            pl.BlockSpec((1, HID // 2), lambda i: (0, 0)),
            pl.BlockSpec((1, 1), lambda i: (0, 0)),
        ],
        out_specs=pl.BlockSpec((BLK, 1), lambda i: (i, 0)),
        out_shape=jax.ShapeDtypeStruct((N, 1), jnp.float32),
        compiler_params=pltpu.CompilerParams(
            dimension_semantics=("arbitrary",)),
    )(agg, yt, deg2, bgc, w1_t, b1r, w2r, b2r)


def kernel(x, edge_index, W_ih, W_hh, b_ih, b_hh, W_gcn, b_gcn, W1, b1, W2, b2):
    src = edge_index[0].astype(jnp.int32)
    dst = edge_index[1].astype(jnp.int32)
    padn = EPAD - E
    src2d = jnp.concatenate(
        [src, jnp.zeros((padn,), jnp.int32)]).reshape(EROWS, 128)
    dst2d = jnp.concatenate(
        [dst, jnp.full((padn,), PAD_DST, jnp.int32)]).reshape(EROWS, 128)

    zeros_deg = jnp.zeros((NPAD, 16), jnp.float32)
    ones_blk = jnp.ones((128, 16), jnp.float32)
    degp = _sc_degree(dst2d, zeros_deg, ones_blk)
    deg2 = (degp[0, :N, 0] + degp[1, :N, 0] + 1.0).reshape(N, 1)

    x2 = x.reshape(N, WIN * IN_CH)
    bias = (b_ih + b_hh).reshape(1, 4 * HID)
    yt = _tc_lstm(x2, W_ih.T, W_hh.T, bias, W_gcn.T, deg2)

    zeros_acc = jnp.zeros((ACC_ROWS, HID), jnp.float32)
    aggp = _sc_gather_scatter(yt, src2d, dst2d, zeros_acc)
    agg = jnp.concatenate([aggp[0, :HALF], aggp[1, :HALF]], axis=0)

    out = _tc_head(agg, yt, deg2, b_gcn.reshape(1, HID), W1.T,
                   b1.reshape(1, HID // 2), W2.reshape(1, HID // 2),
                   b2.reshape(1, 1))
    return out.reshape(1, N, 1)


# speedup vs baseline: 1.4157x; 1.4151x over previous
import functools

import jax
import jax.numpy as jnp
from jax import lax
from jax.experimental import pallas as pl
from jax.experimental.pallas import tpu as pltpu
from jax.experimental.pallas import tpu_sc as plsc

N = 50000
E = 800000
HID = 64
WIN = 8
IN_CH = 8

EPAD = 819200
EROWS = EPAD // 128
NPAD = 50176
HALF = 25000
ACC_ROWS = 25600
DUMP_BASE = 25088
PAD_DST = NPAD - 64
CNT_ROW = N + 16
CAPP = 25600
SEGB = 640

BLK = 1000


def _sc_degree_partition(src2d, dst2d, zeros_deg, ones_blk):
    mesh = plsc.VectorSubcoreMesh(core_axis_name="c", subcore_axis_name="s")
    RPC = EROWS // 2
    RPT = RPC // 16
    ZRT = NPAD // 16
    W = 8

    @functools.partial(
        pl.kernel,
        out_type=(
            jax.ShapeDtypeStruct((2, NPAD, 16), jnp.float32),
            jax.ShapeDtypeStruct((2, 32, CAPP), jnp.int32),
            jax.ShapeDtypeStruct((2, 32, CAPP), jnp.int32),
        ),
        mesh=mesh,
        scratch_types=[
            pltpu.VMEM((RPT, 128), jnp.int32),
            pltpu.VMEM((RPT, 128), jnp.int32),
            pltpu.VMEM((128, 16), jnp.float32),
            [pltpu.VMEM((2, 352), jnp.int32) for _ in range(2)],
            [pltpu.VMEM((2, 352), jnp.int32) for _ in range(2)],
            pltpu.VMEM((1, 16), jnp.float32),
            pltpu.VMEM_SHARED((NPAD, 16), jnp.float32),
            pltpu.SMEM((8,), jnp.int32),
            pltpu.SemaphoreType.DMA,
            [pltpu.SemaphoreType.DMA for _ in range(2)],
        ],
        compiler_params=pltpu.CompilerParams(
            use_tc_tiling_on_sc=False, needs_layout_passes=False),
    )
    def k(dst_hbm, src_hbm, z_hbm, ones_hbm, deg_hbm, ssrc_hbm, sdst_hbm,
          dst_v, src_v, ones_v, sbank, dbank, cnt_v, acc_sh, st, sem, fsem):
        cid = lax.axis_index("c")
        sid = lax.axis_index("s")
        tid = cid * 16 + sid
        lane = lax.iota(jnp.int32, 16)
        pltpu.sync_copy(dst_hbm.at[pl.ds(tid * RPT, RPT)], dst_v)
        pltpu.sync_copy(src_hbm.at[pl.ds(tid * RPT, RPT)], src_v)
        pltpu.sync_copy(ones_hbm, ones_v)
        pltpu.sync_copy(z_hbm.at[pl.ds(sid * ZRT, ZRT)],
                        acc_sh.at[pl.ds(sid * ZRT, ZRT)])
        plsc.subcore_barrier()

        for i in range(8):
            st[i] = 0
        dump_v = DUMP_BASE + (sid & 7) * 64 + lane

        def flush(h):
            b = st[2 + h]
            cc = st[4 + h]

            @pl.when(cc >= 1)
            def _():
                pltpu.make_async_copy(
                    sbank[h].at[0, pl.ds(0, 320)],
                    ssrc_hbm.at[h, tid, pl.ds(0, 320)], fsem[h]).wait()
                pltpu.make_async_copy(
                    dbank[h].at[0, pl.ds(0, 320)],
                    sdst_hbm.at[h, tid, pl.ds(0, 320)], fsem[h]).wait()

            pltpu.async_copy(sbank[h].at[b, pl.ds(0, 320)],
                             ssrc_hbm.at[h, tid, pl.ds(cc * 320, 320)],
                             fsem[h])
            pltpu.async_copy(dbank[h].at[b, pl.ds(0, 320)],
                             sdst_hbm.at[h, tid, pl.ds(cc * 320, 320)],
                             fsem[h])
            nb = 1 - b
            sbank[h].at[nb][pl.ds(0, 16)] = sbank[h].at[b][pl.ds(320, 16)]
            dbank[h].at[nb][pl.ds(0, 16)] = dbank[h].at[b][pl.ds(320, 16)]
            st[2 + h] = nb
            st[4 + h] = cc + 1
            st[h] = st[h] - 320

        @pl.loop(0, RPT)
        def _(j):
            pltpu.async_copy(ones_v, acc_sh.at[dst_v.at[j]], sem, add=True)

            @pl.when(j >= W)
            def _():
                pltpu.make_async_copy(ones_v, acc_sh.at[dst_v.at[0]],
                                      sem).wait()

            for kk in range(8):
                sl = pl.ds(kk * 16, 16)
                d = dst_v.at[j][sl]
                s = src_v.at[j][sl]
                m0 = d < HALF
                m1 = (d >= HALF) & (d < N)
                for h, m, ld in ((0, m0, d), (1, m1, d - HALF)):
                    off = st[h]
                    b = st[2 + h]
                    plsc.store_compressed(
                        sbank[h].at[b, pl.ds(off, 16)], s, mask=m)
                    plsc.store_compressed(
                        dbank[h].at[b, pl.ds(off, 16)], ld, mask=m)
                    st[h] = off + jnp.sum(m.astype(jnp.int32))

                    @pl.when(st[h] >= 320)
                    def _():
                        flush(h)

        for _ in range(W):
            pltpu.make_async_copy(ones_v, acc_sh.at[dst_v.at[0]], sem).wait()

        zeros16 = jnp.zeros((16,), jnp.int32)
        for h in range(2):
            off = st[h]

            @pl.when(off > 0)
            def _():
                b = st[2 + h]
                for i in range(20):
                    @pl.when(off + 16 * i < 320)
                    def _():
                        sbank[h].at[b][pl.ds(off + 16 * i, 16)] = zeros16
                        dbank[h].at[b][pl.ds(off + 16 * i, 16)] = dump_v
                st[h] = 320
                flush(h)

            @pl.when((st[4 + h] & 1) == 1)
            def _():
                b = st[2 + h]
                for i in range(20):
                    sbank[h].at[b][pl.ds(16 * i, 16)] = zeros16
                    dbank[h].at[b][pl.ds(16 * i, 16)] = dump_v
                st[h] = 320
                flush(h)

            cnt_v.at[0][pl.ds(0, 16)] = jnp.where(
                lane == 0, st[4 + h] * 320, 0).astype(jnp.float32)
            pltpu.sync_copy(cnt_v,
                            acc_sh.at[pl.ds(CNT_ROW + 16 * h + sid, 1)])

            @pl.when(st[4 + h] >= 1)
            def _():
                pltpu.make_async_copy(
                    sbank[h].at[0, pl.ds(0, 320)],
                    ssrc_hbm.at[h, tid, pl.ds(0, 320)], fsem[h]).wait()
                pltpu.make_async_copy(
                    dbank[h].at[0, pl.ds(0, 320)],
                    sdst_hbm.at[h, tid, pl.ds(0, 320)], fsem[h]).wait()

        plsc.subcore_barrier()
        pltpu.sync_copy(acc_sh.at[pl.ds(sid * ZRT, ZRT)],
                        deg_hbm.at[cid, pl.ds(sid * ZRT, ZRT)])

    return k(dst2d, src2d, zeros_deg, ones_blk)


def _sc_gather_scatter(yt, seg_src, seg_dst, counts, zeros_acc):
    mesh = plsc.VectorSubcoreMesh(core_axis_name="c", subcore_axis_name="s")
    GW = 64
    NBUF = 5
    D = 3
    ZRT = ACC_ROWS // 16

    @functools.partial(
        pl.kernel,
        out_type=jax.ShapeDtypeStruct((2, ACC_ROWS, HID), jnp.float32),
        mesh=mesh,
        scratch_types=[
            pltpu.VMEM((2, 320), jnp.int32),
            pltpu.VMEM((2, 320), jnp.int32),
            pltpu.VMEM((2, 32), jnp.int32),
            [pltpu.VMEM((1, GW), jnp.int32) for _ in range(NBUF)],
            [pltpu.VMEM((1, GW), jnp.int32) for _ in range(NBUF)],
            [pltpu.VMEM((GW, HID), jnp.float32) for _ in range(NBUF)],
            pltpu.VMEM_SHARED((ACC_ROWS, HID), jnp.float32),
            [pltpu.SemaphoreType.DMA for _ in range(2)],
            [pltpu.SemaphoreType.DMA for _ in range(NBUF)],
            [pltpu.SemaphoreType.DMA for _ in range(NBUF)],
        ],
        compiler_params=pltpu.CompilerParams(
            use_tc_tiling_on_sc=False, needs_layout_passes=False),
    )
    def k(yt_hbm, ssrc_hbm, sdst_hbm, cnt_hbm, z_hbm, out_hbm,
          sbank, dbank, cnt_v, srow, drow, rows_v, acc_sh, isem, gsem, ssem):
        cid = lax.axis_index("c")
        sid = lax.axis_index("s")
        lane = lax.iota(jnp.int32, 16)
        pltpu.sync_copy(z_hbm.at[pl.ds(sid * ZRT, ZRT)],
                        acc_sh.at[pl.ds(sid * ZRT, ZRT)])
        plsc.subcore_barrier()

        pltpu.sync_copy(cnt_hbm, cnt_v)
        cl = cnt_v.at[cid][pl.ds(0, 16)]
        chi = cnt_v.at[cid][pl.ds(16, 16)]
        pick = jnp.where(sid < 8, cl, chi)
        rem = (2 * sid) & 15
        t1 = jnp.sum(jnp.where(lane == rem, pick, 0)) // SEGB
        t2 = jnp.sum(jnp.where(lane == rem + 1, pick, 0)) // SEGB

        def start_bank_load(seg, m, bk):
            pltpu.async_copy(
                ssrc_hbm.at[cid, seg, pl.ds(m * SEGB + bk * 320, 320)],
                sbank.at[bk], isem[bk])
            pltpu.async_copy(
                sdst_hbm.at[cid, seg, pl.ds(m * SEGB + bk * 320, 320)],
                dbank.at[bk], isem[bk])

        def wait_bank_load(seg, m, bk):
            pltpu.make_async_copy(
                ssrc_hbm.at[cid, seg, pl.ds(m * SEGB + bk * 320, 320)],
                sbank.at[bk], isem[bk]).wait()
            pltpu.make_async_copy(
                sdst_hbm.at[cid, seg, pl.ds(m * SEGB + bk * 320, 320)],
                dbank.at[bk], isem[bk]).wait()

        def wait_scatter(q):
            pltpu.make_async_copy(rows_v[q], acc_sh.at[drow[q].at[0]],
                                  ssem[q]).wait()

        def issue_scatter(q):
            pltpu.async_copy(rows_v[q], acc_sh.at[drow[q].at[0]], ssem[q],
                             add=True)

        def wait_gather(q):
            pltpu.make_async_copy(yt_hbm.at[srow[q].at[0]], rows_v[q],
                                  gsem[q]).wait()

        def half_step(bk, u, s, q):
            @pl.when(s >= NBUF)
            def _():
                wait_scatter(q)

            for kk in range(4):
                so = pl.ds(kk * 16, 16)
                si = pl.ds((u % 5) * GW + kk * 16, 16)
                srow[q].at[0][so] = sbank.at[bk][si]
                drow[q].at[0][so] = dbank.at[bk][si]

            pltpu.async_copy(yt_hbm.at[srow[q].at[0]], rows_v[q], gsem[q])

            q2 = (q - D) % NBUF
            @pl.when(s >= D)
            def _():
                wait_gather(q2)
                issue_scatter(q2)

        def run_segment(seg, t_blocks, s_base):
            @pl.when(t_blocks > 0)
            def _():
                start_bank_load(seg, 0, 0)
                start_bank_load(seg, 0, 1)

            @pl.loop(0, t_blocks)
            def _(m):
                wait_bank_load(seg, m, 0)
                for u in range(5):
                    half_step(0, u, s_base + 10 * m + u, u % NBUF)

                @pl.when(m + 1 < t_blocks)
                def _():
                    start_bank_load(seg, m + 1, 0)

                wait_bank_load(seg, m, 1)
                for u in range(5, 10):
                    half_step(1, u, s_base + 10 * m + u, u % NBUF)

                @pl.when(m + 1 < t_blocks)
                def _():
                    start_bank_load(seg, m + 1, 1)

        run_segment(2 * sid, t1, 0)
        run_segment(2 * sid + 1, t2, 10 * t1)

        s_tot = 10 * (t1 + t2)

        @pl.when(s_tot > 0)
        def _():
            for i in range(D):
                q2 = (NBUF - D + i) % NBUF
                wait_gather(q2)
                issue_scatter(q2)
            for q in range(NBUF):
                wait_scatter(q)

        plsc.subcore_barrier()
        pltpu.sync_copy(acc_sh.at[pl.ds(sid * ZRT, ZRT)],
                        out_hbm.at[cid, pl.ds(sid * ZRT, ZRT)])

    return k(yt, seg_src, seg_dst, counts, zeros_acc)


def _tc_lstm(x2, wih_t, whh_t, bias, wg_t, deg2):
    def body(x_ref, wih_ref, whh_ref, b_ref, wg_ref, deg_ref, yt_ref):
        xb = x_ref[...]
        wih = wih_ref[...]
        whh = whh_ref[...]
        b = b_ref[...]
        h = jnp.zeros((BLK, HID), jnp.float32)
        c = jnp.zeros((BLK, HID), jnp.float32)
        for t in range(WIN):
            xt = xb[:, t * IN_CH:(t + 1) * IN_CH]
            g = jnp.dot(xt, wih, preferred_element_type=jnp.float32)
            g = g + jnp.dot(h, whh, preferred_element_type=jnp.float32) + b
            ig = jax.nn.sigmoid(g[:, 0:HID])
            fg = jax.nn.sigmoid(g[:, HID:2 * HID])
            gg = jnp.tanh(g[:, 2 * HID:3 * HID])
            og = jax.nn.sigmoid(g[:, 3 * HID:4 * HID])
            c = fg * c + ig * gg
            h = og * jnp.tanh(c)
        xt_out = jnp.dot(h, wg_ref[...], preferred_element_type=jnp.float32)
        yt_ref[...] = xt_out * lax.rsqrt(deg_ref[...])

    return pl.pallas_call(
        body,
        grid=(N // BLK,),
        in_specs=[
            pl.BlockSpec((BLK, WIN * IN_CH), lambda i: (i, 0)),
            pl.BlockSpec((IN_CH, 4 * HID), lambda i: (0, 0)),
            pl.BlockSpec((HID, 4 * HID), lambda i: (0, 0)),
            pl.BlockSpec((1, 4 * HID), lambda i: (0, 0)),
            pl.BlockSpec((HID, HID), lambda i: (0, 0)),
            pl.BlockSpec((BLK, 1), lambda i: (i, 0)),
        ],
        out_specs=pl.BlockSpec((BLK, HID), lambda i: (i, 0)),
        out_shape=jax.ShapeDtypeStruct((N, HID), jnp.float32),
        compiler_params=pltpu.CompilerParams(
            dimension_semantics=("arbitrary",)),
    )(x2, wih_t, whh_t, bias, wg_t, deg2)


def _tc_head(aggp, yt, deg2, bgc, w1_t, b1r, w2r, b2r):
    def body(a_ref, y_ref, d_ref, bg_ref, w1_ref, b1_ref, w2_ref, b2_ref,
             o_ref):
        dinv = lax.rsqrt(d_ref[...])
        g = dinv * (a_ref[0] + y_ref[...]) + bg_ref[...]
        g = jnp.maximum(g, 0.0)
        o1 = jnp.dot(g, w1_ref[...], preferred_element_type=jnp.float32)
        o1 = jnp.maximum(o1 + b1_ref[...], 0.0)
        o_ref[...] = jnp.sum(o1 * w2_ref[...], axis=1, keepdims=True) + b2_ref[...]

    return pl.pallas_call(
        body,
        grid=(N // BLK,),
        in_specs=[
            pl.BlockSpec((1, BLK, HID), lambda i: (i // 25, i % 25, 0)),
            pl.BlockSpec((BLK, HID), lambda i: (i, 0)),
            pl.BlockSpec((BLK, 1), lambda i: (i, 0)),
            pl.BlockSpec((1, HID), lambda i: (0, 0)),
            pl.BlockSpec((HID, HID // 2), lambda i: (0, 0)),
            pl.BlockSpec((1, HID // 2), lambda i: (0, 0)),
            pl.BlockSpec((1, HID // 2), lambda i: (0, 0)),
            pl.BlockSpec((1, 1), lambda i: (0, 0)),
        ],
        out_specs=pl.BlockSpec((BLK, 1), lambda i: (i, 0)),
        out_shape=jax.ShapeDtypeStruct((N, 1), jnp.float32),
        compiler_params=pltpu.CompilerParams(
            dimension_semantics=("arbitrary",)),
    )(aggp, yt, deg2, bgc, w1_t, b1r, w2r, b2r)


def kernel(x, edge_index, W_ih, W_hh, b_ih, b_hh, W_gcn, b_gcn, W1, b1, W2, b2):
    src = edge_index[0].astype(jnp.int32)
    dst = edge_index[1].astype(jnp.int32)
    padn = EPAD - E
    src2d = jnp.concatenate(
        [src, jnp.zeros((padn,), jnp.int32)]).reshape(EROWS, 128)
    dst2d = jnp.concatenate(
        [dst, jnp.full((padn,), PAD_DST, jnp.int32)]).reshape(EROWS, 128)

    zeros_deg = jnp.zeros((NPAD, 16), jnp.float32)
    ones_blk = jnp.ones((128, 16), jnp.float32)
    degp, seg_src, seg_dst = _sc_degree_partition(
        src2d, dst2d, zeros_deg, ones_blk)
    deg2 = (degp[0, :N, 0] + degp[1, :N, 0] + 1.0).reshape(N, 1)
    counts = jnp.stack([
        jnp.concatenate([degp[0, CNT_ROW + 16 * h:CNT_ROW + 16 * h + 16, 0],
                         degp[1, CNT_ROW + 16 * h:CNT_ROW + 16 * h + 16, 0]])
        for h in range(2)]).astype(jnp.int32)

    x2 = x.reshape(N, WIN * IN_CH)
    bias = (b_ih + b_hh).reshape(1, 4 * HID)
    yt = _tc_lstm(x2, W_ih.T, W_hh.T, bias, W_gcn.T, deg2)

    zeros_acc = jnp.zeros((ACC_ROWS, HID), jnp.float32)
    aggp = _sc_gather_scatter(yt, seg_src, seg_dst, counts, zeros_acc)

    out = _tc_head(aggp, yt, deg2, b_gcn.reshape(1, HID), W1.T,
                   b1.reshape(1, HID // 2), W2.reshape(1, HID // 2),
                   b2.reshape(1, 1))
    return out.reshape(1, N, 1)


# speedup vs baseline: 1.5243x; 1.0767x over previous
import functools

import jax
import jax.numpy as jnp
from jax import lax
from jax.experimental import pallas as pl
from jax.experimental.pallas import tpu as pltpu
from jax.experimental.pallas import tpu_sc as plsc

N = 50000
E = 800000
HID = 64
WIN = 8
IN_CH = 8

EPAD = 819200
EROWS = EPAD // 128
NPAD = 50176
HALF = 25000
ACC_ROWS = 25600
DUMP_BASE = 25088
PAD_DST = NPAD - 64
CNT_ROW = N + 16
CAPP = 25600
SEGB = 640

BLK = 5000
BPP = HALF // BLK


def _sc_degree_partition(src2d, dst2d, zeros_deg, ones_blk):
    mesh = plsc.VectorSubcoreMesh(core_axis_name="c", subcore_axis_name="s")
    RPC = EROWS // 2
    RPT = RPC // 16
    ZRT = NPAD // 16
    W = 8

    @functools.partial(
        pl.kernel,
        out_type=(
            jax.ShapeDtypeStruct((2, NPAD, 16), jnp.float32),
            jax.ShapeDtypeStruct((2, 32, CAPP), jnp.int32),
            jax.ShapeDtypeStruct((2, 32, CAPP), jnp.int32),
        ),
        mesh=mesh,
        scratch_types=[
            pltpu.VMEM((RPT, 128), jnp.int32),
            pltpu.VMEM((RPT, 128), jnp.int32),
            pltpu.VMEM((128, 16), jnp.float32),
            [pltpu.VMEM((2, 352), jnp.int32) for _ in range(2)],
            [pltpu.VMEM((2, 352), jnp.int32) for _ in range(2)],
            pltpu.VMEM((1, 16), jnp.float32),
            pltpu.VMEM_SHARED((NPAD, 16), jnp.float32),
            pltpu.SMEM((8,), jnp.int32),
            pltpu.SemaphoreType.DMA,
            [pltpu.SemaphoreType.DMA for _ in range(2)],
        ],
        compiler_params=pltpu.CompilerParams(
            use_tc_tiling_on_sc=False, needs_layout_passes=False),
    )
    def k(dst_hbm, src_hbm, z_hbm, ones_hbm, deg_hbm, ssrc_hbm, sdst_hbm,
          dst_v, src_v, ones_v, sbank, dbank, cnt_v, acc_sh, st, sem, fsem):
        cid = lax.axis_index("c")
        sid = lax.axis_index("s")
        tid = cid * 16 + sid
        lane = lax.iota(jnp.int32, 16)
        pltpu.sync_copy(dst_hbm.at[pl.ds(tid * RPT, RPT)], dst_v)
        pltpu.sync_copy(src_hbm.at[pl.ds(tid * RPT, RPT)], src_v)
        pltpu.sync_copy(ones_hbm, ones_v)
        pltpu.sync_copy(z_hbm.at[pl.ds(sid * ZRT, ZRT)],
                        acc_sh.at[pl.ds(sid * ZRT, ZRT)])
        plsc.subcore_barrier()

        for i in range(8):
            st[i] = 0
        dump_v = DUMP_BASE + (sid & 7) * 64 + lane

        def flush(h):
            b = st[2 + h]
            cc = st[4 + h]

            @pl.when(cc >= 1)
            def _():
                pltpu.make_async_copy(
                    sbank[h].at[0, pl.ds(0, 320)],
                    ssrc_hbm.at[h, tid, pl.ds(0, 320)], fsem[h]).wait()
                pltpu.make_async_copy(
                    dbank[h].at[0, pl.ds(0, 320)],
                    sdst_hbm.at[h, tid, pl.ds(0, 320)], fsem[h]).wait()

            pltpu.async_copy(sbank[h].at[b, pl.ds(0, 320)],
                             ssrc_hbm.at[h, tid, pl.ds(cc * 320, 320)],
                             fsem[h])
            pltpu.async_copy(dbank[h].at[b, pl.ds(0, 320)],
                             sdst_hbm.at[h, tid, pl.ds(cc * 320, 320)],
                             fsem[h])
            nb = 1 - b
            sbank[h].at[nb][pl.ds(0, 16)] = sbank[h].at[b][pl.ds(320, 16)]
            dbank[h].at[nb][pl.ds(0, 16)] = dbank[h].at[b][pl.ds(320, 16)]
            st[2 + h] = nb
            st[4 + h] = cc + 1
            st[h] = st[h] - 320

        @pl.loop(0, RPT)
        def _(j):
            pltpu.async_copy(ones_v, acc_sh.at[dst_v.at[j]], sem, add=True)

            @pl.when(j >= W)
            def _():
                pltpu.make_async_copy(ones_v, acc_sh.at[dst_v.at[0]],
                                      sem).wait()

            for kk in range(8):
                sl = pl.ds(kk * 16, 16)
                d = dst_v.at[j][sl]
                s = src_v.at[j][sl]
                m0 = d < HALF
                m1 = (d >= HALF) & (d < N)
                for h, m, ld in ((0, m0, d), (1, m1, d - HALF)):
                    off = st[h]
                    b = st[2 + h]
                    plsc.store_compressed(
                        sbank[h].at[b, pl.ds(off, 16)], s, mask=m)
                    plsc.store_compressed(
                        dbank[h].at[b, pl.ds(off, 16)], ld, mask=m)
                    st[h] = off + jnp.sum(m.astype(jnp.int32))

                    @pl.when(st[h] >= 320)
                    def _():
                        flush(h)

        for _ in range(W):
            pltpu.make_async_copy(ones_v, acc_sh.at[dst_v.at[0]], sem).wait()

        zeros16 = jnp.zeros((16,), jnp.int32)
        for h in range(2):
            off = st[h]

            @pl.when(off > 0)
            def _():
                b = st[2 + h]
                for i in range(20):
                    @pl.when(off + 16 * i < 320)
                    def _():
                        sbank[h].at[b][pl.ds(off + 16 * i, 16)] = zeros16
                        dbank[h].at[b][pl.ds(off + 16 * i, 16)] = dump_v
                st[h] = 320
                flush(h)

            @pl.when((st[4 + h] & 1) == 1)
            def _():
                b = st[2 + h]
                for i in range(20):
                    sbank[h].at[b][pl.ds(16 * i, 16)] = zeros16
                    dbank[h].at[b][pl.ds(16 * i, 16)] = dump_v
                st[h] = 320
                flush(h)

            cnt_v.at[0][pl.ds(0, 16)] = jnp.where(
                lane == 0, st[4 + h] * 320, 0).astype(jnp.float32)
            pltpu.sync_copy(cnt_v,
                            acc_sh.at[pl.ds(CNT_ROW + 16 * h + sid, 1)])

            @pl.when(st[4 + h] >= 1)
            def _():
                pltpu.make_async_copy(
                    sbank[h].at[0, pl.ds(0, 320)],
                    ssrc_hbm.at[h, tid, pl.ds(0, 320)], fsem[h]).wait()
                pltpu.make_async_copy(
                    dbank[h].at[0, pl.ds(0, 320)],
                    sdst_hbm.at[h, tid, pl.ds(0, 320)], fsem[h]).wait()

        plsc.subcore_barrier()
        pltpu.sync_copy(acc_sh.at[pl.ds(sid * ZRT, ZRT)],
                        deg_hbm.at[cid, pl.ds(sid * ZRT, ZRT)])

    return k(dst2d, src2d, zeros_deg, ones_blk)


def _sc_gather_scatter(yt, seg_src, seg_dst, counts, zeros_acc):
    mesh = plsc.VectorSubcoreMesh(core_axis_name="c", subcore_axis_name="s")
    GW = 64
    NBUF = 5
    D = 4
    ZRT = ACC_ROWS // 16

    @functools.partial(
        pl.kernel,
        out_type=jax.ShapeDtypeStruct((2, ACC_ROWS, HID), jnp.float32),
        mesh=mesh,
        scratch_types=[
            pltpu.VMEM((2, 320), jnp.int32),
            pltpu.VMEM((2, 320), jnp.int32),
            pltpu.VMEM((2, 32), jnp.int32),
            [pltpu.VMEM((1, GW), jnp.int32) for _ in range(NBUF)],
            [pltpu.VMEM((1, GW), jnp.int32) for _ in range(NBUF)],
            [pltpu.VMEM((GW, HID), jnp.float32) for _ in range(NBUF)],
            pltpu.VMEM_SHARED((ACC_ROWS, HID), jnp.float32),
            [pltpu.SemaphoreType.DMA for _ in range(2)],
            [pltpu.SemaphoreType.DMA for _ in range(NBUF)],
            [pltpu.SemaphoreType.DMA for _ in range(NBUF)],
        ],
        compiler_params=pltpu.CompilerParams(
            use_tc_tiling_on_sc=False, needs_layout_passes=False),
    )
    def k(yt_hbm, ssrc_hbm, sdst_hbm, cnt_hbm, z_hbm, out_hbm,
          sbank, dbank, cnt_v, srow, drow, rows_v, acc_sh, isem, gsem, ssem):
        cid = lax.axis_index("c")
        sid = lax.axis_index("s")
        lane = lax.iota(jnp.int32, 16)
        pltpu.sync_copy(z_hbm.at[pl.ds(sid * ZRT, ZRT)],
                        acc_sh.at[pl.ds(sid * ZRT, ZRT)])
        plsc.subcore_barrier()

        pltpu.sync_copy(cnt_hbm, cnt_v)
        cl = cnt_v.at[cid][pl.ds(0, 16)]
        chi = cnt_v.at[cid][pl.ds(16, 16)]
        pick = jnp.where(sid < 8, cl, chi)
        rem = (2 * sid) & 15
        t1 = jnp.sum(jnp.where(lane == rem, pick, 0)) // SEGB
        t2 = jnp.sum(jnp.where(lane == rem + 1, pick, 0)) // SEGB

        def start_bank_load(seg, m, bk):
            pltpu.async_copy(
                ssrc_hbm.at[cid, seg, pl.ds(m * SEGB + bk * 320, 320)],
                sbank.at[bk], isem[bk])
            pltpu.async_copy(
                sdst_hbm.at[cid, seg, pl.ds(m * SEGB + bk * 320, 320)],
                dbank.at[bk], isem[bk])

        def wait_bank_load(seg, m, bk):
            pltpu.make_async_copy(
                ssrc_hbm.at[cid, seg, pl.ds(m * SEGB + bk * 320, 320)],
                sbank.at[bk], isem[bk]).wait()
            pltpu.make_async_copy(
                sdst_hbm.at[cid, seg, pl.ds(m * SEGB + bk * 320, 320)],
                dbank.at[bk], isem[bk]).wait()

        def wait_scatter(q):
            pltpu.make_async_copy(rows_v[q], acc_sh.at[drow[q].at[0]],
                                  ssem[q]).wait()

        def issue_scatter(q):
            pltpu.async_copy(rows_v[q], acc_sh.at[drow[q].at[0]], ssem[q],
                             add=True)

        def wait_gather(q):
            pltpu.make_async_copy(yt_hbm.at[srow[q].at[0]], rows_v[q],
                                  gsem[q]).wait()

        def half_step(bk, u, s, q):
            @pl.when(s >= NBUF)
            def _():
                wait_scatter(q)

            for kk in range(4):
                so = pl.ds(kk * 16, 16)
                si = pl.ds((u % 5) * GW + kk * 16, 16)
                srow[q].at[0][so] = sbank.at[bk][si]
                drow[q].at[0][so] = dbank.at[bk][si]

            pltpu.async_copy(yt_hbm.at[srow[q].at[0]], rows_v[q], gsem[q])

            q2 = (q - D) % NBUF
            @pl.when(s >= D)
            def _():
                wait_gather(q2)
                issue_scatter(q2)

        def run_segment(seg, t_blocks, s_base):
            @pl.when(t_blocks > 0)
            def _():
                start_bank_load(seg, 0, 0)
                start_bank_load(seg, 0, 1)

            @pl.loop(0, t_blocks)
            def _(m):
                wait_bank_load(seg, m, 0)
                for u in range(5):
                    half_step(0, u, s_base + 10 * m + u, u % NBUF)

                @pl.when(m + 1 < t_blocks)
                def _():
                    start_bank_load(seg, m + 1, 0)

                wait_bank_load(seg, m, 1)
                for u in range(5, 10):
                    half_step(1, u, s_base + 10 * m + u, u % NBUF)

                @pl.when(m + 1 < t_blocks)
                def _():
                    start_bank_load(seg, m + 1, 1)

        run_segment(2 * sid, t1, 0)
        run_segment(2 * sid + 1, t2, 10 * t1)

        s_tot = 10 * (t1 + t2)

        @pl.when(s_tot > 0)
        def _():
            for i in range(D):
                q2 = (NBUF - D + i) % NBUF
                wait_gather(q2)
                issue_scatter(q2)
            for q in range(NBUF):
                wait_scatter(q)

        plsc.subcore_barrier()
        pltpu.sync_copy(acc_sh.at[pl.ds(sid * ZRT, ZRT)],
                        out_hbm.at[cid, pl.ds(sid * ZRT, ZRT)])

    return k(yt, seg_src, seg_dst, counts, zeros_acc)


def _tc_lstm(x2, wih_t, whh_t, bias, wg_t, deg2):
    def body(x_ref, wih_ref, whh_ref, b_ref, wg_ref, deg_ref, yt_ref):
        xb = x_ref[...]
        wih = wih_ref[...]
        whh = whh_ref[...]
        b = b_ref[...]
        h = jnp.zeros((BLK, HID), jnp.float32)
        c = jnp.zeros((BLK, HID), jnp.float32)
        for t in range(WIN):
            xt = xb[:, t * IN_CH:(t + 1) * IN_CH]
            g = jnp.dot(xt, wih, preferred_element_type=jnp.float32)
            g = g + jnp.dot(h, whh, preferred_element_type=jnp.float32) + b
            ig = jax.nn.sigmoid(g[:, 0:HID])
            fg = jax.nn.sigmoid(g[:, HID:2 * HID])
            gg = jnp.tanh(g[:, 2 * HID:3 * HID])
            og = jax.nn.sigmoid(g[:, 3 * HID:4 * HID])
            c = fg * c + ig * gg
            h = og * jnp.tanh(c)
        xt_out = jnp.dot(h, wg_ref[...], preferred_element_type=jnp.float32)
        yt_ref[...] = xt_out * lax.rsqrt(deg_ref[...])

    return pl.pallas_call(
        body,
        grid=(N // BLK,),
        in_specs=[
            pl.BlockSpec((BLK, WIN * IN_CH), lambda i: (i, 0)),
            pl.BlockSpec((IN_CH, 4 * HID), lambda i: (0, 0)),
            pl.BlockSpec((HID, 4 * HID), lambda i: (0, 0)),
            pl.BlockSpec((1, 4 * HID), lambda i: (0, 0)),
            pl.BlockSpec((HID, HID), lambda i: (0, 0)),
            pl.BlockSpec((BLK, 1), lambda i: (i, 0)),
        ],
        out_specs=pl.BlockSpec((BLK, HID), lambda i: (i, 0)),
        out_shape=jax.ShapeDtypeStruct((N, HID), jnp.float32),
        compiler_params=pltpu.CompilerParams(
            dimension_semantics=("arbitrary",)),
    )(x2, wih_t, whh_t, bias, wg_t, deg2)


def _tc_head(aggp, yt, deg2, bgc, w1_t, b1r, w2r, b2r):
    def body(a_ref, y_ref, d_ref, bg_ref, w1_ref, b1_ref, w2_ref, b2_ref,
             o_ref):
        dinv = lax.rsqrt(d_ref[...])
        g = dinv * (a_ref[0] + y_ref[...]) + bg_ref[...]
        g = jnp.maximum(g, 0.0)
        o1 = jnp.dot(g, w1_ref[...], preferred_element_type=jnp.float32)
        o1 = jnp.maximum(o1 + b1_ref[...], 0.0)
        o_ref[...] = jnp.sum(o1 * w2_ref[...], axis=1, keepdims=True) + b2_ref[...]

    return pl.pallas_call(
        body,
        grid=(N // BLK,),
        in_specs=[
            pl.BlockSpec((1, BLK, HID), lambda i: (i // BPP, i % BPP, 0)),
            pl.BlockSpec((BLK, HID), lambda i: (i, 0)),
            pl.BlockSpec((BLK, 1), lambda i: (i, 0)),
            pl.BlockSpec((1, HID), lambda i: (0, 0)),
            pl.BlockSpec((HID, HID // 2), lambda i: (0, 0)),
            pl.BlockSpec((1, HID // 2), lambda i: (0, 0)),
            pl.BlockSpec((1, HID // 2), lambda i: (0, 0)),
            pl.BlockSpec((1, 1), lambda i: (0, 0)),
        ],
        out_specs=pl.BlockSpec((BLK, 1), lambda i: (i, 0)),
        out_shape=jax.ShapeDtypeStruct((N, 1), jnp.float32),
        compiler_params=pltpu.CompilerParams(
            dimension_semantics=("arbitrary",)),
    )(aggp, yt, deg2, bgc, w1_t, b1r, w2r, b2r)


def kernel(x, edge_index, W_ih, W_hh, b_ih, b_hh, W_gcn, b_gcn, W1, b1, W2, b2):
    src = edge_index[0].astype(jnp.int32)
    dst = edge_index[1].astype(jnp.int32)
    padn = EPAD - E
    src2d = jnp.concatenate(
        [src, jnp.zeros((padn,), jnp.int32)]).reshape(EROWS, 128)
    dst2d = jnp.concatenate(
        [dst, jnp.full((padn,), PAD_DST, jnp.int32)]).reshape(EROWS, 128)

    zeros_deg = jnp.zeros((NPAD, 16), jnp.float32)
    ones_blk = jnp.ones((128, 16), jnp.float32)
    degp, seg_src, seg_dst = _sc_degree_partition(
        src2d, dst2d, zeros_deg, ones_blk)
    deg2 = (degp[0, :N, 0] + degp[1, :N, 0] + 1.0).reshape(N, 1)
    counts = jnp.stack([
        jnp.concatenate([degp[0, CNT_ROW + 16 * h:CNT_ROW + 16 * h + 16, 0],
                         degp[1, CNT_ROW + 16 * h:CNT_ROW + 16 * h + 16, 0]])
        for h in range(2)]).astype(jnp.int32)

    x2 = x.reshape(N, WIN * IN_CH)
    bias = (b_ih + b_hh).reshape(1, 4 * HID)
    yt = _tc_lstm(x2, W_ih.T, W_hh.T, bias, W_gcn.T, deg2)

    zeros_acc = jnp.zeros((ACC_ROWS, HID), jnp.float32)
    aggp = _sc_gather_scatter(yt, seg_src, seg_dst, counts, zeros_acc)

    out = _tc_head(aggp, yt, deg2, b_gcn.reshape(1, HID), W1.T,
                   b1.reshape(1, HID // 2), W2.reshape(1, HID // 2),
                   b2.reshape(1, 1))
    return out.reshape(1, N, 1)


# speedup vs baseline: 1.8219x; 1.1952x over previous
import functools

import jax
import jax.numpy as jnp
from jax import lax
from jax.experimental import pallas as pl
from jax.experimental.pallas import tpu as pltpu
from jax.experimental.pallas import tpu_sc as plsc

N = 50000
E = 800000
HID = 64
WIN = 8
IN_CH = 8

EPAD = 819200
EROWS = EPAD // 128
NPAD = 50176
HALF = 25000
ACC_ROWS = 25600
DUMP_BASE = 25088
PAD_DST = NPAD - 64
CNT_ROW = N + 16
CAPP = 25600
SEGB = 640

BLK = 5000
BPP = HALF // BLK


def _sc_degree_partition(src2d, dst2d, zeros_deg, ones_blk):
    mesh = plsc.VectorSubcoreMesh(core_axis_name="c", subcore_axis_name="s")
    RPC = EROWS // 2
    RPT = RPC // 16
    ZRT = NPAD // 16
    W = 8

    @functools.partial(
        pl.kernel,
        out_type=(
            jax.ShapeDtypeStruct((2, NPAD, 16), jnp.float32),
            jax.ShapeDtypeStruct((2, 32, CAPP), jnp.int32),
            jax.ShapeDtypeStruct((2, 32, CAPP), jnp.int32),
        ),
        mesh=mesh,
        scratch_types=[
            pltpu.VMEM((RPT, 128), jnp.int32),
            pltpu.VMEM((RPT, 128), jnp.int32),
            pltpu.VMEM((128, 16), jnp.float32),
            [pltpu.VMEM((2, 352), jnp.int32) for _ in range(2)],
            [pltpu.VMEM((2, 352), jnp.int32) for _ in range(2)],
            pltpu.VMEM((1, 16), jnp.float32),
            pltpu.VMEM_SHARED((NPAD, 16), jnp.float32),
            pltpu.SMEM((8,), jnp.int32),
            pltpu.SemaphoreType.DMA,
            [pltpu.SemaphoreType.DMA for _ in range(2)],
        ],
        compiler_params=pltpu.CompilerParams(
            use_tc_tiling_on_sc=False, needs_layout_passes=False),
    )
    def k(dst_hbm, src_hbm, z_hbm, ones_hbm, deg_hbm, ssrc_hbm, sdst_hbm,
          dst_v, src_v, ones_v, sbank, dbank, cnt_v, acc_sh, st, sem, fsem):
        cid = lax.axis_index("c")
        sid = lax.axis_index("s")
        tid = cid * 16 + sid
        lane = lax.iota(jnp.int32, 16)
        pltpu.sync_copy(dst_hbm.at[pl.ds(tid * RPT, RPT)], dst_v)
        pltpu.sync_copy(src_hbm.at[pl.ds(tid * RPT, RPT)], src_v)
        pltpu.sync_copy(ones_hbm, ones_v)
        pltpu.sync_copy(z_hbm.at[pl.ds(sid * ZRT, ZRT)],
                        acc_sh.at[pl.ds(sid * ZRT, ZRT)])
        plsc.subcore_barrier()

        for i in range(8):
            st[i] = 0
        dump_v = DUMP_BASE + (sid & 7) * 64 + lane

        def flush(h):
            b = st[2 + h]
            cc = st[4 + h]

            @pl.when(cc >= 1)
            def _():
                pltpu.make_async_copy(
                    sbank[h].at[0, pl.ds(0, 320)],
                    ssrc_hbm.at[h, tid, pl.ds(0, 320)], fsem[h]).wait()
                pltpu.make_async_copy(
                    dbank[h].at[0, pl.ds(0, 320)],
                    sdst_hbm.at[h, tid, pl.ds(0, 320)], fsem[h]).wait()

            pltpu.async_copy(sbank[h].at[b, pl.ds(0, 320)],
                             ssrc_hbm.at[h, tid, pl.ds(cc * 320, 320)],
                             fsem[h])
            pltpu.async_copy(dbank[h].at[b, pl.ds(0, 320)],
                             sdst_hbm.at[h, tid, pl.ds(cc * 320, 320)],
                             fsem[h])
            nb = 1 - b
            sbank[h].at[nb][pl.ds(0, 16)] = sbank[h].at[b][pl.ds(320, 16)]
            dbank[h].at[nb][pl.ds(0, 16)] = dbank[h].at[b][pl.ds(320, 16)]
            st[2 + h] = nb
            st[4 + h] = cc + 1
            st[h] = st[h] - 320

        @pl.loop(0, RPT)
        def _(j):
            pltpu.async_copy(ones_v, acc_sh.at[dst_v.at[j]], sem, add=True)

            @pl.when(j >= W)
            def _():
                pltpu.make_async_copy(ones_v, acc_sh.at[dst_v.at[0]],
                                      sem).wait()

            for kk in range(8):
                sl = pl.ds(kk * 16, 16)
                d = dst_v.at[j][sl]
                s = src_v.at[j][sl]
                m0 = d < HALF
                m1 = (d >= HALF) & (d < N)
                for h, m, ld in ((0, m0, d), (1, m1, d - HALF)):
                    off = st[h]
                    b = st[2 + h]
                    plsc.store_compressed(
                        sbank[h].at[b, pl.ds(off, 16)], s, mask=m)
                    plsc.store_compressed(
                        dbank[h].at[b, pl.ds(off, 16)], ld, mask=m)
                    st[h] = off + jnp.sum(m.astype(jnp.int32))

                    @pl.when(st[h] >= 320)
                    def _():
                        flush(h)

        for _ in range(W):
            pltpu.make_async_copy(ones_v, acc_sh.at[dst_v.at[0]], sem).wait()

        zeros16 = jnp.zeros((16,), jnp.int32)
        for h in range(2):
            off = st[h]

            @pl.when(off > 0)
            def _():
                b = st[2 + h]
                for i in range(20):
                    @pl.when(off + 16 * i < 320)
                    def _():
                        sbank[h].at[b][pl.ds(off + 16 * i, 16)] = zeros16
                        dbank[h].at[b][pl.ds(off + 16 * i, 16)] = dump_v
                st[h] = 320
                flush(h)

            @pl.when((st[4 + h] & 1) == 1)
            def _():
                b = st[2 + h]
                for i in range(20):
                    sbank[h].at[b][pl.ds(16 * i, 16)] = zeros16
                    dbank[h].at[b][pl.ds(16 * i, 16)] = dump_v
                st[h] = 320
                flush(h)

            cnt_v.at[0][pl.ds(0, 16)] = jnp.where(
                lane == 0, st[4 + h] * 320, 0).astype(jnp.float32)
            pltpu.sync_copy(cnt_v,
                            acc_sh.at[pl.ds(CNT_ROW + 16 * h + sid, 1)])

            @pl.when(st[4 + h] >= 1)
            def _():
                pltpu.make_async_copy(
                    sbank[h].at[0, pl.ds(0, 320)],
                    ssrc_hbm.at[h, tid, pl.ds(0, 320)], fsem[h]).wait()
                pltpu.make_async_copy(
                    dbank[h].at[0, pl.ds(0, 320)],
                    sdst_hbm.at[h, tid, pl.ds(0, 320)], fsem[h]).wait()

        plsc.subcore_barrier()
        pltpu.sync_copy(acc_sh.at[pl.ds(sid * ZRT, ZRT)],
                        deg_hbm.at[cid, pl.ds(sid * ZRT, ZRT)])

    return k(dst2d, src2d, zeros_deg, ones_blk)


def _sc_gather_scatter(yt, seg_src, seg_dst, counts, zeros_acc):
    mesh = plsc.VectorSubcoreMesh(core_axis_name="c", subcore_axis_name="s")
    GW = 64
    NBUF = 5
    D = 4
    ZRT = ACC_ROWS // 16

    @functools.partial(
        pl.kernel,
        out_type=jax.ShapeDtypeStruct((2, ACC_ROWS, HID), jnp.bfloat16),
        mesh=mesh,
        scratch_types=[
            pltpu.VMEM((2, 320), jnp.int32),
            pltpu.VMEM((2, 320), jnp.int32),
            pltpu.VMEM((2, 32), jnp.int32),
            [pltpu.VMEM((1, GW), jnp.int32) for _ in range(NBUF)],
            [pltpu.VMEM((1, GW), jnp.int32) for _ in range(NBUF)],
            [pltpu.VMEM((GW, HID), jnp.bfloat16) for _ in range(NBUF)],
            pltpu.VMEM_SHARED((ACC_ROWS, HID), jnp.bfloat16),
            [pltpu.SemaphoreType.DMA for _ in range(2)],
            [pltpu.SemaphoreType.DMA for _ in range(NBUF)],
            [pltpu.SemaphoreType.DMA for _ in range(NBUF)],
        ],
        compiler_params=pltpu.CompilerParams(
            use_tc_tiling_on_sc=False, needs_layout_passes=False),
    )
    def k(yt_hbm, ssrc_hbm, sdst_hbm, cnt_hbm, z_hbm, out_hbm,
          sbank, dbank, cnt_v, srow, drow, rows_v, acc_sh, isem, gsem, ssem):
        cid = lax.axis_index("c")
        sid = lax.axis_index("s")
        lane = lax.iota(jnp.int32, 16)
        pltpu.sync_copy(z_hbm.at[pl.ds(sid * ZRT, ZRT)],
                        acc_sh.at[pl.ds(sid * ZRT, ZRT)])
        plsc.subcore_barrier()

        pltpu.sync_copy(cnt_hbm, cnt_v)
        cl = cnt_v.at[cid][pl.ds(0, 16)]
        chi = cnt_v.at[cid][pl.ds(16, 16)]
        pick = jnp.where(sid < 8, cl, chi)
        rem = (2 * sid) & 15
        t1 = jnp.sum(jnp.where(lane == rem, pick, 0)) // SEGB
        t2 = jnp.sum(jnp.where(lane == rem + 1, pick, 0)) // SEGB

        def start_bank_load(seg, m, bk):
            pltpu.async_copy(
                ssrc_hbm.at[cid, seg, pl.ds(m * SEGB + bk * 320, 320)],
                sbank.at[bk], isem[bk])
            pltpu.async_copy(
                sdst_hbm.at[cid, seg, pl.ds(m * SEGB + bk * 320, 320)],
                dbank.at[bk], isem[bk])

        def wait_bank_load(seg, m, bk):
            pltpu.make_async_copy(
                ssrc_hbm.at[cid, seg, pl.ds(m * SEGB + bk * 320, 320)],
                sbank.at[bk], isem[bk]).wait()
            pltpu.make_async_copy(
                sdst_hbm.at[cid, seg, pl.ds(m * SEGB + bk * 320, 320)],
                dbank.at[bk], isem[bk]).wait()

        def wait_scatter(q):
            pltpu.make_async_copy(rows_v[q], acc_sh.at[drow[q].at[0]],
                                  ssem[q]).wait()

        def issue_scatter(q):
            pltpu.async_copy(rows_v[q], acc_sh.at[drow[q].at[0]], ssem[q],
                             add=True)

        def wait_gather(q):
            pltpu.make_async_copy(yt_hbm.at[srow[q].at[0]], rows_v[q],
                                  gsem[q]).wait()

        def half_step(bk, u, s, q):
            @pl.when(s >= NBUF)
            def _():
                wait_scatter(q)

            for kk in range(4):
                so = pl.ds(kk * 16, 16)
                si = pl.ds((u % 5) * GW + kk * 16, 16)
                srow[q].at[0][so] = sbank.at[bk][si]
                drow[q].at[0][so] = dbank.at[bk][si]

            pltpu.async_copy(yt_hbm.at[srow[q].at[0]], rows_v[q], gsem[q])

            q2 = (q - D) % NBUF
            @pl.when(s >= D)
            def _():
                wait_gather(q2)
                issue_scatter(q2)

        def run_segment(seg, t_blocks, s_base):
            @pl.when(t_blocks > 0)
            def _():
                start_bank_load(seg, 0, 0)
                start_bank_load(seg, 0, 1)

            @pl.loop(0, t_blocks)
            def _(m):
                wait_bank_load(seg, m, 0)
                for u in range(5):
                    half_step(0, u, s_base + 10 * m + u, u % NBUF)

                @pl.when(m + 1 < t_blocks)
                def _():
                    start_bank_load(seg, m + 1, 0)

                wait_bank_load(seg, m, 1)
                for u in range(5, 10):
                    half_step(1, u, s_base + 10 * m + u, u % NBUF)

                @pl.when(m + 1 < t_blocks)
                def _():
                    start_bank_load(seg, m + 1, 1)

        run_segment(2 * sid, t1, 0)
        run_segment(2 * sid + 1, t2, 10 * t1)

        s_tot = 10 * (t1 + t2)

        @pl.when(s_tot > 0)
        def _():
            for i in range(D):
                q2 = (NBUF - D + i) % NBUF
                wait_gather(q2)
                issue_scatter(q2)
            for q in range(NBUF):
                wait_scatter(q)

        plsc.subcore_barrier()
        pltpu.sync_copy(acc_sh.at[pl.ds(sid * ZRT, ZRT)],
                        out_hbm.at[cid, pl.ds(sid * ZRT, ZRT)])

    return k(yt, seg_src, seg_dst, counts, zeros_acc)


def _tc_lstm(x2, wih_t, whh_t, bias, wg_t, deg2):
    def body(x_ref, wih_ref, whh_ref, b_ref, wg_ref, deg_ref, yt_ref,
             yt16_ref):
        xb = x_ref[...]
        wih = wih_ref[...]
        whh = whh_ref[...]
        b = b_ref[...]
        h = jnp.zeros((BLK, HID), jnp.float32)
        c = jnp.zeros((BLK, HID), jnp.float32)
        for t in range(WIN):
            xt = xb[:, t * IN_CH:(t + 1) * IN_CH]
            g = jnp.dot(xt, wih, preferred_element_type=jnp.float32)
            g = g + jnp.dot(h, whh, preferred_element_type=jnp.float32) + b
            ig = jax.nn.sigmoid(g[:, 0:HID])
            fg = jax.nn.sigmoid(g[:, HID:2 * HID])
            gg = jnp.tanh(g[:, 2 * HID:3 * HID])
            og = jax.nn.sigmoid(g[:, 3 * HID:4 * HID])
            c = fg * c + ig * gg
            h = og * jnp.tanh(c)
        xt_out = jnp.dot(h, wg_ref[...], preferred_element_type=jnp.float32)
        yt = xt_out * lax.rsqrt(deg_ref[...])
        yt_ref[...] = yt
        yt16_ref[...] = yt.astype(jnp.bfloat16)

    return pl.pallas_call(
        body,
        grid=(N // BLK,),
        in_specs=[
            pl.BlockSpec((BLK, WIN * IN_CH), lambda i: (i, 0)),
            pl.BlockSpec((IN_CH, 4 * HID), lambda i: (0, 0)),
            pl.BlockSpec((HID, 4 * HID), lambda i: (0, 0)),
            pl.BlockSpec((1, 4 * HID), lambda i: (0, 0)),
            pl.BlockSpec((HID, HID), lambda i: (0, 0)),
            pl.BlockSpec((BLK, 1), lambda i: (i, 0)),
        ],
        out_specs=[pl.BlockSpec((BLK, HID), lambda i: (i, 0)),
                   pl.BlockSpec((BLK, HID), lambda i: (i, 0))],
        out_shape=[jax.ShapeDtypeStruct((N, HID), jnp.float32),
                   jax.ShapeDtypeStruct((N, HID), jnp.bfloat16)],
        compiler_params=pltpu.CompilerParams(
            dimension_semantics=("arbitrary",)),
    )(x2, wih_t, whh_t, bias, wg_t, deg2)


def _tc_head(aggp, yt, deg2, bgc, w1_t, b1r, w2r, b2r):
    def body(a_ref, y_ref, d_ref, bg_ref, w1_ref, b1_ref, w2_ref, b2_ref,
             o_ref):
        dinv = lax.rsqrt(d_ref[...])
        g = dinv * (a_ref[0].astype(jnp.float32) + y_ref[...]) + bg_ref[...]
        g = jnp.maximum(g, 0.0)
        o1 = jnp.dot(g, w1_ref[...], preferred_element_type=jnp.float32)
        o1 = jnp.maximum(o1 + b1_ref[...], 0.0)
        o_ref[...] = jnp.sum(o1 * w2_ref[...], axis=1, keepdims=True) + b2_ref[...]

    return pl.pallas_call(
        body,
        grid=(N // BLK,),
        in_specs=[
            pl.BlockSpec((1, BLK, HID), lambda i: (i // BPP, i % BPP, 0)),
            pl.BlockSpec((BLK, HID), lambda i: (i, 0)),
            pl.BlockSpec((BLK, 1), lambda i: (i, 0)),
            pl.BlockSpec((1, HID), lambda i: (0, 0)),
            pl.BlockSpec((HID, HID // 2), lambda i: (0, 0)),
            pl.BlockSpec((1, HID // 2), lambda i: (0, 0)),
            pl.BlockSpec((1, HID // 2), lambda i: (0, 0)),
            pl.BlockSpec((1, 1), lambda i: (0, 0)),
        ],
        out_specs=pl.BlockSpec((BLK, 1), lambda i: (i, 0)),
        out_shape=jax.ShapeDtypeStruct((N, 1), jnp.float32),
        compiler_params=pltpu.CompilerParams(
            dimension_semantics=("arbitrary",)),
    )(aggp, yt, deg2, bgc, w1_t, b1r, w2r, b2r)


def kernel(x, edge_index, W_ih, W_hh, b_ih, b_hh, W_gcn, b_gcn, W1, b1, W2, b2):
    src = edge_index[0].astype(jnp.int32)
    dst = edge_index[1].astype(jnp.int32)
    padn = EPAD - E
    src2d = jnp.concatenate(
        [src, jnp.zeros((padn,), jnp.int32)]).reshape(EROWS, 128)
    dst2d = jnp.concatenate(
        [dst, jnp.full((padn,), PAD_DST, jnp.int32)]).reshape(EROWS, 128)

    zeros_deg = jnp.zeros((NPAD, 16), jnp.float32)
    ones_blk = jnp.ones((128, 16), jnp.float32)
    degp, seg_src, seg_dst = _sc_degree_partition(
        src2d, dst2d, zeros_deg, ones_blk)
    deg2 = (degp[0, :N, 0] + degp[1, :N, 0] + 1.0).reshape(N, 1)
    counts = jnp.stack([
        jnp.concatenate([degp[0, CNT_ROW + 16 * h:CNT_ROW + 16 * h + 16, 0],
                         degp[1, CNT_ROW + 16 * h:CNT_ROW + 16 * h + 16, 0]])
        for h in range(2)]).astype(jnp.int32)

    x2 = x.reshape(N, WIN * IN_CH)
    bias = (b_ih + b_hh).reshape(1, 4 * HID)
    yt, yt16 = _tc_lstm(x2, W_ih.T, W_hh.T, bias, W_gcn.T, deg2)

    zeros_acc = jnp.zeros((ACC_ROWS, HID), jnp.bfloat16)
    aggp = _sc_gather_scatter(yt16, seg_src, seg_dst, counts, zeros_acc)

    out = _tc_head(aggp, yt, deg2, b_gcn.reshape(1, HID), W1.T,
                   b1.reshape(1, HID // 2), W2.reshape(1, HID // 2),
                   b2.reshape(1, 1))
    return out.reshape(1, N, 1)


# speedup vs baseline: 1.8943x; 1.0397x over previous
import functools

import jax
import jax.numpy as jnp
from jax import lax
from jax.experimental import pallas as pl
from jax.experimental.pallas import tpu as pltpu
from jax.experimental.pallas import tpu_sc as plsc

N = 50000
E = 800000
HID = 64
WIN = 8
IN_CH = 8

EPAD = 819200
EROWS = EPAD // 128
NPAD = 50176
HALF = 25000
ACC_ROWS = 25600
DUMP_BASE = 25088
PAD_DST = NPAD - 64
CNT_ROW = N + 16
CAPP = 25600
SEGB = 640

BLK = 5000
BPP = HALF // BLK


def _sc_degree_partition(src2d, dst2d, zeros_deg, ones_blk):
    mesh = plsc.VectorSubcoreMesh(core_axis_name="c", subcore_axis_name="s")
    RPC = EROWS // 2
    RPT = RPC // 16
    ZRT = NPAD // 16
    W = 8

    @functools.partial(
        pl.kernel,
        out_type=(
            jax.ShapeDtypeStruct((2, NPAD, 16), jnp.float32),
            jax.ShapeDtypeStruct((2, 32, CAPP), jnp.int32),
            jax.ShapeDtypeStruct((2, 32, CAPP), jnp.int32),
        ),
        mesh=mesh,
        scratch_types=[
            pltpu.VMEM((RPT, 128), jnp.int32),
            pltpu.VMEM((RPT, 128), jnp.int32),
            pltpu.VMEM((128, 16), jnp.float32),
            [pltpu.VMEM((2, 352), jnp.int32) for _ in range(2)],
            [pltpu.VMEM((2, 352), jnp.int32) for _ in range(2)],
            pltpu.VMEM((1, 16), jnp.float32),
            pltpu.VMEM_SHARED((NPAD, 16), jnp.float32),
            pltpu.SMEM((8,), jnp.int32),
            pltpu.SemaphoreType.DMA,
            [pltpu.SemaphoreType.DMA for _ in range(2)],
        ],
        compiler_params=pltpu.CompilerParams(
            use_tc_tiling_on_sc=False, needs_layout_passes=False),
    )
    def k(dst_hbm, src_hbm, z_hbm, ones_hbm, deg_hbm, ssrc_hbm, sdst_hbm,
          dst_v, src_v, ones_v, sbank, dbank, cnt_v, acc_sh, st, sem, fsem):
        cid = lax.axis_index("c")
        sid = lax.axis_index("s")
        tid = cid * 16 + sid
        lane = lax.iota(jnp.int32, 16)
        pltpu.sync_copy(dst_hbm.at[pl.ds(tid * RPT, RPT)], dst_v)
        pltpu.sync_copy(src_hbm.at[pl.ds(tid * RPT, RPT)], src_v)
        pltpu.sync_copy(ones_hbm, ones_v)
        pltpu.sync_copy(z_hbm.at[pl.ds(sid * ZRT, ZRT)],
                        acc_sh.at[pl.ds(sid * ZRT, ZRT)])
        plsc.subcore_barrier()

        for i in range(8):
            st[i] = 0
        dump_v = DUMP_BASE + (sid & 7) * 64 + lane

        def flush(h):
            b = st[2 + h]
            cc = st[4 + h]

            @pl.when(cc >= 1)
            def _():
                pltpu.make_async_copy(
                    sbank[h].at[0, pl.ds(0, 320)],
                    ssrc_hbm.at[h, tid, pl.ds(0, 320)], fsem[h]).wait()
                pltpu.make_async_copy(
                    dbank[h].at[0, pl.ds(0, 320)],
                    sdst_hbm.at[h, tid, pl.ds(0, 320)], fsem[h]).wait()

            pltpu.async_copy(sbank[h].at[b, pl.ds(0, 320)],
                             ssrc_hbm.at[h, tid, pl.ds(cc * 320, 320)],
                             fsem[h])
            pltpu.async_copy(dbank[h].at[b, pl.ds(0, 320)],
                             sdst_hbm.at[h, tid, pl.ds(cc * 320, 320)],
                             fsem[h])
            nb = 1 - b
            sbank[h].at[nb][pl.ds(0, 16)] = sbank[h].at[b][pl.ds(320, 16)]
            dbank[h].at[nb][pl.ds(0, 16)] = dbank[h].at[b][pl.ds(320, 16)]
            st[2 + h] = nb
            st[4 + h] = cc + 1
            st[h] = st[h] - 320

        @pl.loop(0, RPT)
        def _(j):
            pltpu.async_copy(ones_v, acc_sh.at[dst_v.at[j]], sem, add=True)

            @pl.when(j >= W)
            def _():
                pltpu.make_async_copy(ones_v, acc_sh.at[dst_v.at[0]],
                                      sem).wait()

            for kk in range(8):
                sl = pl.ds(kk * 16, 16)
                d = dst_v.at[j][sl]
                s = src_v.at[j][sl]
                m0 = d < HALF
                m1 = (d >= HALF) & (d < N)
                for h, m, ld in ((0, m0, d), (1, m1, d - HALF)):
                    off = st[h]
                    b = st[2 + h]
                    plsc.store_compressed(
                        sbank[h].at[b, pl.ds(off, 16)], s, mask=m)
                    plsc.store_compressed(
                        dbank[h].at[b, pl.ds(off, 16)], ld, mask=m)
                    st[h] = off + jnp.sum(m.astype(jnp.int32))

                    @pl.when(st[h] >= 320)
                    def _():
                        flush(h)

        for _ in range(W):
            pltpu.make_async_copy(ones_v, acc_sh.at[dst_v.at[0]], sem).wait()

        zeros16 = jnp.zeros((16,), jnp.int32)
        for h in range(2):
            off = st[h]

            @pl.when(off > 0)
            def _():
                b = st[2 + h]
                for i in range(20):
                    @pl.when(off + 16 * i < 320)
                    def _():
                        sbank[h].at[b][pl.ds(off + 16 * i, 16)] = zeros16
                        dbank[h].at[b][pl.ds(off + 16 * i, 16)] = dump_v
                st[h] = 320
                flush(h)

            @pl.when((st[4 + h] & 1) == 1)
            def _():
                b = st[2 + h]
                for i in range(20):
                    sbank[h].at[b][pl.ds(16 * i, 16)] = zeros16
                    dbank[h].at[b][pl.ds(16 * i, 16)] = dump_v
                st[h] = 320
                flush(h)

            cnt_v.at[0][pl.ds(0, 16)] = jnp.where(
                lane == 0, st[4 + h] * 320, 0).astype(jnp.float32)
            pltpu.sync_copy(cnt_v,
                            acc_sh.at[pl.ds(CNT_ROW + 16 * h + sid, 1)])

            @pl.when(st[4 + h] >= 1)
            def _():
                pltpu.make_async_copy(
                    sbank[h].at[0, pl.ds(0, 320)],
                    ssrc_hbm.at[h, tid, pl.ds(0, 320)], fsem[h]).wait()
                pltpu.make_async_copy(
                    dbank[h].at[0, pl.ds(0, 320)],
                    sdst_hbm.at[h, tid, pl.ds(0, 320)], fsem[h]).wait()

        plsc.subcore_barrier()
        pltpu.sync_copy(acc_sh.at[pl.ds(sid * ZRT, ZRT)],
                        deg_hbm.at[cid, pl.ds(sid * ZRT, ZRT)])

    return k(dst2d, src2d, zeros_deg, ones_blk)


def _sc_gather_scatter(yt, seg_src, seg_dst, counts, zeros_acc):
    mesh = plsc.VectorSubcoreMesh(core_axis_name="c", subcore_axis_name="s")
    GW = 64
    NBUF = 5
    D = 4
    ZRT = ACC_ROWS // 16

    @functools.partial(
        pl.kernel,
        out_type=jax.ShapeDtypeStruct((2, ACC_ROWS, HID), jnp.bfloat16),
        mesh=mesh,
        scratch_types=[
            pltpu.VMEM((2, 320), jnp.int32),
            pltpu.VMEM((2, 320), jnp.int32),
            pltpu.VMEM((2, 32), jnp.int32),
            [pltpu.VMEM((1, GW), jnp.int32) for _ in range(NBUF)],
            [pltpu.VMEM((1, GW), jnp.int32) for _ in range(NBUF)],
            [pltpu.VMEM((GW, HID), jnp.bfloat16) for _ in range(NBUF)],
            pltpu.VMEM_SHARED((ACC_ROWS, HID), jnp.bfloat16),
            [pltpu.SemaphoreType.DMA for _ in range(2)],
            [pltpu.SemaphoreType.DMA for _ in range(NBUF)],
            [pltpu.SemaphoreType.DMA for _ in range(NBUF)],
        ],
        compiler_params=pltpu.CompilerParams(
            use_tc_tiling_on_sc=False, needs_layout_passes=False),
    )
    def k(yt_hbm, ssrc_hbm, sdst_hbm, cnt_hbm, z_hbm, out_hbm,
          sbank, dbank, cnt_v, srow, drow, rows_v, acc_sh, isem, gsem, ssem):
        cid = lax.axis_index("c")
        sid = lax.axis_index("s")
        lane = lax.iota(jnp.int32, 16)
        pltpu.sync_copy(z_hbm.at[pl.ds(sid * ZRT, ZRT)],
                        acc_sh.at[pl.ds(sid * ZRT, ZRT)])
        plsc.subcore_barrier()

        pltpu.sync_copy(cnt_hbm, cnt_v)
        cl = cnt_v.at[cid][pl.ds(0, 16)]
        chi = cnt_v.at[cid][pl.ds(16, 16)]
        pick = jnp.where(sid < 8, cl, chi)
        rem = (2 * sid) & 15
        t1 = jnp.sum(jnp.where(lane == rem, pick, 0)) // SEGB
        t2 = jnp.sum(jnp.where(lane == rem + 1, pick, 0)) // SEGB

        def start_bank_load(seg, m, bk):
            pltpu.async_copy(
                ssrc_hbm.at[cid, seg, pl.ds(m * SEGB + bk * 320, 320)],
                sbank.at[bk], isem[bk])
            pltpu.async_copy(
                sdst_hbm.at[cid, seg, pl.ds(m * SEGB + bk * 320, 320)],
                dbank.at[bk], isem[bk])

        def wait_bank_load(seg, m, bk):
            pltpu.make_async_copy(
                ssrc_hbm.at[cid, seg, pl.ds(m * SEGB + bk * 320, 320)],
                sbank.at[bk], isem[bk]).wait()
            pltpu.make_async_copy(
                sdst_hbm.at[cid, seg, pl.ds(m * SEGB + bk * 320, 320)],
                dbank.at[bk], isem[bk]).wait()

        def wait_scatter(q):
            pltpu.make_async_copy(rows_v[q], acc_sh.at[drow[q].at[0]],
                                  ssem[q]).wait()

        def issue_scatter(q):
            pltpu.async_copy(rows_v[q], acc_sh.at[drow[q].at[0]], ssem[q],
                             add=True)

        def wait_gather(q):
            pltpu.make_async_copy(yt_hbm.at[srow[q].at[0]], rows_v[q],
                                  gsem[q]).wait()

        def half_step(bk, u, s, q):
            @pl.when(s >= NBUF)
            def _():
                wait_scatter(q)

            for kk in range(4):
                so = pl.ds(kk * 16, 16)
                si = pl.ds((u % 5) * GW + kk * 16, 16)
                srow[q].at[0][so] = sbank.at[bk][si]
                drow[q].at[0][so] = dbank.at[bk][si]

            pltpu.async_copy(yt_hbm.at[srow[q].at[0]], rows_v[q], gsem[q])

            q2 = (q - D) % NBUF
            @pl.when(s >= D)
            def _():
                wait_gather(q2)
                issue_scatter(q2)

        def run_segment(seg, t_blocks, s_base):
            @pl.when(t_blocks > 0)
            def _():
                start_bank_load(seg, 0, 0)
                start_bank_load(seg, 0, 1)

            @pl.loop(0, t_blocks)
            def _(m):
                wait_bank_load(seg, m, 0)
                for u in range(5):
                    half_step(0, u, s_base + 10 * m + u, u % NBUF)

                @pl.when(m + 1 < t_blocks)
                def _():
                    start_bank_load(seg, m + 1, 0)

                wait_bank_load(seg, m, 1)
                for u in range(5, 10):
                    half_step(1, u, s_base + 10 * m + u, u % NBUF)

                @pl.when(m + 1 < t_blocks)
                def _():
                    start_bank_load(seg, m + 1, 1)

        run_segment(2 * sid, t1, 0)
        run_segment(2 * sid + 1, t2, 10 * t1)

        s_tot = 10 * (t1 + t2)

        @pl.when(s_tot > 0)
        def _():
            for i in range(D):
                q2 = (NBUF - D + i) % NBUF
                wait_gather(q2)
                issue_scatter(q2)
            for q in range(NBUF):
                wait_scatter(q)

        plsc.subcore_barrier()
        pltpu.sync_copy(acc_sh.at[pl.ds(sid * ZRT, ZRT)],
                        out_hbm.at[cid, pl.ds(sid * ZRT, ZRT)])

    return k(yt, seg_src, seg_dst, counts, zeros_acc)


def _tc_lstm(x2, wih_t, whh_t, bias, wg_t, deg2):
    def body(x_ref, wih_ref, whh_ref, b_ref, wg_ref, deg_ref, yt_ref,
             yt16_ref):
        xb = x_ref[...]
        wih = wih_ref[...]
        whh = whh_ref[...]
        b = b_ref[...]
        h = jnp.zeros((BLK, HID), jnp.float32)
        c = jnp.zeros((BLK, HID), jnp.float32)
        for t in range(WIN):
            xt = xb[:, t * IN_CH:(t + 1) * IN_CH]
            g = jnp.dot(xt, wih, preferred_element_type=jnp.float32)
            g = g + jnp.dot(h, whh, preferred_element_type=jnp.float32) + b
            ig = 0.5 * jnp.tanh(0.5 * g[:, 0:HID]) + 0.5
            fg = 0.5 * jnp.tanh(0.5 * g[:, HID:2 * HID]) + 0.5
            gg = jnp.tanh(g[:, 2 * HID:3 * HID])
            og = 0.5 * jnp.tanh(0.5 * g[:, 3 * HID:4 * HID]) + 0.5
            c = fg * c + ig * gg
            h = og * jnp.tanh(c)
        xt_out = jnp.dot(h, wg_ref[...], preferred_element_type=jnp.float32)
        yt = xt_out * lax.rsqrt(deg_ref[...])
        yt_ref[...] = yt
        yt16_ref[...] = yt.astype(jnp.bfloat16)

    return pl.pallas_call(
        body,
        grid=(N // BLK,),
        in_specs=[
            pl.BlockSpec((BLK, WIN * IN_CH), lambda i: (i, 0)),
            pl.BlockSpec((IN_CH, 4 * HID), lambda i: (0, 0)),
            pl.BlockSpec((HID, 4 * HID), lambda i: (0, 0)),
            pl.BlockSpec((1, 4 * HID), lambda i: (0, 0)),
            pl.BlockSpec((HID, HID), lambda i: (0, 0)),
            pl.BlockSpec((BLK, 1), lambda i: (i, 0)),
        ],
        out_specs=[pl.BlockSpec((BLK, HID), lambda i: (i, 0)),
                   pl.BlockSpec((BLK, HID), lambda i: (i, 0))],
        out_shape=[jax.ShapeDtypeStruct((N, HID), jnp.float32),
                   jax.ShapeDtypeStruct((N, HID), jnp.bfloat16)],
        compiler_params=pltpu.CompilerParams(
            dimension_semantics=("arbitrary",)),
    )(x2, wih_t, whh_t, bias, wg_t, deg2)


def _tc_head(aggp, yt, deg2, bgc, w1_t, b1r, w2r, b2r):
    def body(a_ref, y_ref, d_ref, bg_ref, w1_ref, b1_ref, w2_ref, b2_ref,
             o_ref):
        dinv = lax.rsqrt(d_ref[...])
        g = dinv * (a_ref[0].astype(jnp.float32) + y_ref[...]) + bg_ref[...]
        g = jnp.maximum(g, 0.0)
        o1 = jnp.dot(g, w1_ref[...], preferred_element_type=jnp.float32)
        o1 = jnp.maximum(o1 + b1_ref[...], 0.0)
        o_ref[...] = jnp.sum(o1 * w2_ref[...], axis=1, keepdims=True) + b2_ref[...]

    return pl.pallas_call(
        body,
        grid=(N // BLK,),
        in_specs=[
            pl.BlockSpec((1, BLK, HID), lambda i: (i // BPP, i % BPP, 0)),
            pl.BlockSpec((BLK, HID), lambda i: (i, 0)),
            pl.BlockSpec((BLK, 1), lambda i: (i, 0)),
            pl.BlockSpec((1, HID), lambda i: (0, 0)),
            pl.BlockSpec((HID, HID // 2), lambda i: (0, 0)),
            pl.BlockSpec((1, HID // 2), lambda i: (0, 0)),
            pl.BlockSpec((1, HID // 2), lambda i: (0, 0)),
            pl.BlockSpec((1, 1), lambda i: (0, 0)),
        ],
        out_specs=pl.BlockSpec((BLK, 1), lambda i: (i, 0)),
        out_shape=jax.ShapeDtypeStruct((N, 1), jnp.float32),
        compiler_params=pltpu.CompilerParams(
            dimension_semantics=("arbitrary",)),
    )(aggp, yt, deg2, bgc, w1_t, b1r, w2r, b2r)


def kernel(x, edge_index, W_ih, W_hh, b_ih, b_hh, W_gcn, b_gcn, W1, b1, W2, b2):
    src = edge_index[0].astype(jnp.int32)
    dst = edge_index[1].astype(jnp.int32)
    padn = EPAD - E
    src2d = jnp.concatenate(
        [src, jnp.zeros((padn,), jnp.int32)]).reshape(EROWS, 128)
    dst2d = jnp.concatenate(
        [dst, jnp.full((padn,), PAD_DST, jnp.int32)]).reshape(EROWS, 128)

    zeros_deg = jnp.zeros((NPAD, 16), jnp.float32)
    ones_blk = jnp.ones((128, 16), jnp.float32)
    degp, seg_src, seg_dst = _sc_degree_partition(
        src2d, dst2d, zeros_deg, ones_blk)
    deg2 = (degp[0, :N, 0] + degp[1, :N, 0] + 1.0).reshape(N, 1)
    counts = jnp.stack([
        jnp.concatenate([degp[0, CNT_ROW + 16 * h:CNT_ROW + 16 * h + 16, 0],
                         degp[1, CNT_ROW + 16 * h:CNT_ROW + 16 * h + 16, 0]])
        for h in range(2)]).astype(jnp.int32)

    x2 = x.reshape(N, WIN * IN_CH)
    bias = (b_ih + b_hh).reshape(1, 4 * HID)
    yt, yt16 = _tc_lstm(x2, W_ih.T, W_hh.T, bias, W_gcn.T, deg2)

    zeros_acc = jnp.zeros((ACC_ROWS, HID), jnp.bfloat16)
    aggp = _sc_gather_scatter(yt16, seg_src, seg_dst, counts, zeros_acc)

    out = _tc_head(aggp, yt, deg2, b_gcn.reshape(1, HID), W1.T,
                   b1.reshape(1, HID // 2), W2.reshape(1, HID // 2),
                   b2.reshape(1, 1))
    return out.reshape(1, N, 1)
